# baseline (speedup 1.0000x reference)
import functools

import jax
import jax.numpy as jnp
from jax import lax
from jax.experimental import pallas as pl
from jax.experimental.pallas import tpu as pltpu
from jax.experimental.pallas import tpu_sc as plsc

N = 10000
NFEAT = 128
NHID = 128
NCLASS = 40
E = 320000

NC = 2
NS = 16
L = 16
NW = NC * NS

IDX_ROWS_PER_TILE = 80
E_PAD = NW * IDX_ROWS_PER_TILE * 128
GROUPS = 5
G = IDX_ROWS_PER_TILE // GROUPS
NP = 10240
RPT = NP // NS
D2P = 48

_mesh = plsc.VectorSubcoreMesh(core_axis_name="c", subcore_axis_name="s")


@functools.partial(
    pl.kernel,
    out_type=jax.ShapeDtypeStruct((NC, NP, L), jnp.float32),
    mesh=_mesh,
    compiler_params=pltpu.CompilerParams(use_tc_tiling_on_sc=False),
    scratch_types=[
        pltpu.VMEM((G, 128), jnp.int32),
        pltpu.VMEM((128, L), jnp.float32),
        pltpu.VMEM_SHARED((NP, L), jnp.float32),
    ],
)
def _sc_degree(col_hbm, out_hbm, cidx, buf, acc):
    c = lax.axis_index("c")
    s = lax.axis_index("s")
    wid = c * NS + s

    @pl.loop(0, 128)
    def _(i):
        buf[i, :] = jnp.zeros((L,), jnp.float32)

    @pl.loop(0, RPT // 128)
    def _(z):
        pltpu.sync_copy(buf, acc.at[pl.ds(s * RPT + z * 128, 128)])

    @pl.loop(0, 128)
    def _(i):
        buf[i, :] = jnp.full((L,), 1.0, jnp.float32)

    plsc.subcore_barrier()

    @pl.loop(0, GROUPS)
    def _(t):
        pltpu.sync_copy(
            col_hbm.at[pl.ds(wid * IDX_ROWS_PER_TILE + t * G, G)], cidx)
        for jj in range(G):
            pltpu.sync_copy(buf, acc.at[cidx.at[jj]], add=True)

    plsc.subcore_barrier()
    pltpu.sync_copy(acc.at[pl.ds(s * RPT, RPT)],
                    out_hbm.at[c, pl.ds(s * RPT, RPT)])


def _make_prop(D):
    @functools.partial(
        pl.kernel,
        out_type=jax.ShapeDtypeStruct((NC, NP, D), jnp.float32),
        mesh=_mesh,
        compiler_params=pltpu.CompilerParams(
            use_tc_tiling_on_sc=(D % 128 == 0)),
        scratch_types=[
            pltpu.VMEM((G, 128), jnp.int32),
            pltpu.VMEM((G, 128), jnp.int32),
            pltpu.VMEM((128, D), jnp.float32),
            pltpu.VMEM_SHARED((NP, D), jnp.float32),
        ],
    )
    def _prop(g_hbm, row_hbm, col_hbm, out_hbm, ridx, cidx, rows, acc):
        c = lax.axis_index("c")
        s = lax.axis_index("s")
        wid = c * NS + s

        pltpu.sync_copy(g_hbm.at[pl.ds(s * RPT, RPT)],
                        acc.at[pl.ds(s * RPT, RPT)])
        plsc.subcore_barrier()

        @pl.loop(0, GROUPS)
        def _(t):
            base = wid * IDX_ROWS_PER_TILE + t * G
            pltpu.sync_copy(row_hbm.at[pl.ds(base, G)], ridx)
            pltpu.sync_copy(col_hbm.at[pl.ds(base, G)], cidx)
            for jj in range(G):
                pltpu.sync_copy(g_hbm.at[ridx.at[jj]], rows)
                pltpu.sync_copy(rows, acc.at[cidx.at[jj]], add=True)

        plsc.subcore_barrier()
        pltpu.sync_copy(acc.at[pl.ds(s * RPT, RPT)],
                        out_hbm.at[c, pl.ds(s * RPT, RPT)])

    return _prop


_prop128 = _make_prop(NHID)
_prop48 = _make_prop(D2P)

BLK = 1024


def _tc_pre_body(d0_ref, d1_ref, x_ref, w_ref, g_ref, dinv_ref):
    deg = d0_ref[...][:, 0:1] + d1_ref[...][:, 0:1] + 1.0
    dinv = lax.rsqrt(deg)
    h = jnp.dot(x_ref[...], w_ref[...], preferred_element_type=jnp.float32)
    g_ref[...] = h * dinv
    dinv_ref[...] = dinv


_tc_pre = pl.pallas_call(
    _tc_pre_body,
    grid=(NP // BLK,),
    in_specs=[
        pl.BlockSpec((BLK, L), lambda i: (i, 0)),
        pl.BlockSpec((BLK, L), lambda i: (i, 0)),
        pl.BlockSpec((BLK, NFEAT), lambda i: (i, 0)),
        pl.BlockSpec((NFEAT, NHID), lambda i: (0, 0)),
    ],
    out_specs=[
        pl.BlockSpec((BLK, NHID), lambda i: (i, 0)),
        pl.BlockSpec((BLK, 1), lambda i: (i, 0)),
    ],
    out_shape=[
        jax.ShapeDtypeStruct((NP, NHID), jnp.float32),
        jax.ShapeDtypeStruct((NP, 1), jnp.float32),
    ],
)


def _tc_mid_body(a_ref, g1_ref, dinv_ref, b1_ref, w2_ref, g2_ref):
    dinv = dinv_ref[...]
    x1 = jnp.maximum(
        dinv * (a_ref[0] + a_ref[1] - g1_ref[...]) + b1_ref[...], 0.0)
    g2_ref[...] = dinv * jnp.dot(
        x1, w2_ref[...], preferred_element_type=jnp.float32)


_tc_mid = pl.pallas_call(
    _tc_mid_body,
    grid=(NP // BLK,),
    in_specs=[
        pl.BlockSpec((NC, BLK, NHID), lambda i: (0, i, 0)),
        pl.BlockSpec((BLK, NHID), lambda i: (i, 0)),
        pl.BlockSpec((BLK, 1), lambda i: (i, 0)),
        pl.BlockSpec((1, NHID), lambda i: (0, 0)),
        pl.BlockSpec((NHID, D2P), lambda i: (0, 0)),
    ],
    out_specs=pl.BlockSpec((BLK, D2P), lambda i: (i, 0)),
    out_shape=jax.ShapeDtypeStruct((NP, D2P), jnp.float32),
)


def _tc_post_body(a_ref, g2_ref, dinv_ref, b2_ref, o_ref):
    dinv = dinv_ref[...]
    o_ref[...] = dinv * (a_ref[0] + a_ref[1] - g2_ref[...]) + b2_ref[...]


_tc_post = pl.pallas_call(
    _tc_post_body,
    grid=(NP // BLK,),
    in_specs=[
        pl.BlockSpec((NC, BLK, D2P), lambda i: (0, i, 0)),
        pl.BlockSpec((BLK, D2P), lambda i: (i, 0)),
        pl.BlockSpec((BLK, 1), lambda i: (i, 0)),
        pl.BlockSpec((1, D2P), lambda i: (0, 0)),
    ],
    out_specs=pl.BlockSpec((BLK, D2P), lambda i: (i, 0)),
    out_shape=jax.ShapeDtypeStruct((NP, D2P), jnp.float32),
)


def kernel(x, adj, W1, b1, W2, b2):
    row = adj[0].astype(jnp.int32)
    col = adj[1].astype(jnp.int32)
    pad = E_PAD - E
    rowp = jnp.concatenate(
        [row, jnp.zeros((pad,), jnp.int32)]).reshape(E_PAD // 128, 128)
    padcol = N + jnp.arange(pad, dtype=jnp.int32) % (NP - N)
    colp = jnp.concatenate([col, padcol]).reshape(E_PAD // 128, 128)
    xp = jnp.pad(x, ((0, NP - N), (0, 0)))

    degp = _sc_degree(colp)
    g1, dinv = _tc_pre(degp[0], degp[1], xp, W1)
    acc1 = _prop128(g1, rowp, colp)

    W2p = jnp.pad(W2, ((0, 0), (0, D2P - NCLASS)))
    b1r = b1.reshape(1, NHID)
    b2p = jnp.pad(b2, (0, D2P - NCLASS)).reshape(1, D2P)

    g2 = _tc_mid(acc1, g1, dinv, b1r, W2p)
    acc2 = _prop48(g2, rowp, colp)
    out = _tc_post(acc2, g2, dinv, b2p)
    return out[:N, :NCLASS]


# speedup vs baseline: 1.3852x; 1.3852x over previous
import functools

import jax
import jax.numpy as jnp
from jax import lax
from jax.experimental import pallas as pl
from jax.experimental.pallas import tpu as pltpu
from jax.experimental.pallas import tpu_sc as plsc

N = 10000
NFEAT = 128
NHID = 128
NCLASS = 40
E = 320000

NC = 2
NS = 16
L = 16
NW = NC * NS

IDX_ROWS_PER_TILE = 80
E_PAD = NW * IDX_ROWS_PER_TILE * 128
GROUPS = 5
G = IDX_ROWS_PER_TILE // GROUPS
NP = 10240
RPT = NP // NS
D2P = 48

_mesh = plsc.VectorSubcoreMesh(core_axis_name="c", subcore_axis_name="s")


@functools.partial(
    pl.kernel,
    out_type=jax.ShapeDtypeStruct((NC, NP, L), jnp.float32),
    mesh=_mesh,
    compiler_params=pltpu.CompilerParams(use_tc_tiling_on_sc=False),
    scratch_types=[
        pltpu.VMEM((G, 128), jnp.int32),
        pltpu.VMEM((128, L), jnp.float32),
        pltpu.VMEM_SHARED((NP, L), jnp.float32),
    ],
)
def _sc_degree(col_hbm, out_hbm, cidx, buf, acc):
    c = lax.axis_index("c")
    s = lax.axis_index("s")
    wid = c * NS + s

    @pl.loop(0, 128)
    def _(i):
        buf[i, :] = jnp.zeros((L,), jnp.float32)

    @pl.loop(0, RPT // 128)
    def _(z):
        pltpu.sync_copy(buf, acc.at[pl.ds(s * RPT + z * 128, 128)])

    @pl.loop(0, 128)
    def _(i):
        buf[i, :] = jnp.full((L,), 1.0, jnp.float32)

    plsc.subcore_barrier()

    @pl.loop(0, GROUPS)
    def _(t):
        pltpu.sync_copy(
            col_hbm.at[pl.ds(wid * IDX_ROWS_PER_TILE + t * G, G)], cidx)
        for jj in range(G):
            pltpu.sync_copy(buf, acc.at[cidx.at[jj]], add=True)

    plsc.subcore_barrier()
    pltpu.sync_copy(acc.at[pl.ds(s * RPT, RPT)],
                    out_hbm.at[c, pl.ds(s * RPT, RPT)])


def _make_prop(D):
    T = IDX_ROWS_PER_TILE
    H = T // 2

    @functools.partial(
        pl.kernel,
        out_type=jax.ShapeDtypeStruct((NC, NP, D), jnp.float32),
        mesh=_mesh,
        compiler_params=pltpu.CompilerParams(
            use_tc_tiling_on_sc=(D % 128 == 0)),
        scratch_types=[
            pltpu.VMEM((H, 128), jnp.int32),
            pltpu.VMEM((H, 128), jnp.int32),
            pltpu.VMEM((128, D), jnp.float32),
            pltpu.VMEM((128, D), jnp.float32),
            pltpu.SemaphoreType.DMA,
            pltpu.SemaphoreType.DMA,
            pltpu.VMEM_SHARED((NP, D), jnp.float32),
        ],
    )
    def _prop(g_hbm, row_hbm, col_hbm, out_hbm, ridx, cidx, bufa, bufb,
              sema, semb, acc):
        c = lax.axis_index("c")
        s = lax.axis_index("s")
        wid = c * NS + s

        pltpu.sync_copy(g_hbm.at[pl.ds(s * RPT, RPT)],
                        acc.at[pl.ds(s * RPT, RPT)])
        plsc.subcore_barrier()

        for h in range(2):
            base = wid * T + h * H
            pltpu.sync_copy(row_hbm.at[pl.ds(base, H)], ridx)
            pltpu.sync_copy(col_hbm.at[pl.ds(base, H)], cidx)
            pltpu.async_copy(g_hbm.at[ridx.at[0]], bufa, sema)

            @pl.loop(0, H, step=2)
            def _(t):
                pltpu.async_copy(g_hbm.at[ridx.at[t + 1]], bufb, semb)
                pltpu.make_async_copy(g_hbm.at[ridx.at[t]], bufa, sema).wait()
                pltpu.sync_copy(bufa, acc.at[cidx.at[t]], add=True)
                tn = jnp.minimum(t + 2, H - 1)
                pltpu.async_copy(g_hbm.at[ridx.at[tn]], bufa, sema)
                pltpu.make_async_copy(
                    g_hbm.at[ridx.at[t + 1]], bufb, semb).wait()
                pltpu.sync_copy(bufb, acc.at[cidx.at[t + 1]], add=True)

            pltpu.make_async_copy(g_hbm.at[ridx.at[H - 1]], bufa, sema).wait()

        plsc.subcore_barrier()
        pltpu.sync_copy(acc.at[pl.ds(s * RPT, RPT)],
                        out_hbm.at[c, pl.ds(s * RPT, RPT)])

    return _prop


_prop128 = _make_prop(NHID)
_prop48 = _make_prop(D2P)

BLK = 1024


def _tc_pre_body(d0_ref, d1_ref, x_ref, w_ref, g_ref, dinv_ref):
    deg = d0_ref[...][:, 0:1] + d1_ref[...][:, 0:1] + 1.0
    dinv = lax.rsqrt(deg)
    h = jnp.dot(x_ref[...], w_ref[...], preferred_element_type=jnp.float32)
    g_ref[...] = h * dinv
    dinv_ref[...] = dinv


_tc_pre = pl.pallas_call(
    _tc_pre_body,
    grid=(NP // BLK,),
    in_specs=[
        pl.BlockSpec((BLK, L), lambda i: (i, 0)),
        pl.BlockSpec((BLK, L), lambda i: (i, 0)),
        pl.BlockSpec((BLK, NFEAT), lambda i: (i, 0)),
        pl.BlockSpec((NFEAT, NHID), lambda i: (0, 0)),
    ],
    out_specs=[
        pl.BlockSpec((BLK, NHID), lambda i: (i, 0)),
        pl.BlockSpec((BLK, 1), lambda i: (i, 0)),
    ],
    out_shape=[
        jax.ShapeDtypeStruct((NP, NHID), jnp.float32),
        jax.ShapeDtypeStruct((NP, 1), jnp.float32),
    ],
)


def _tc_mid_body(a_ref, g1_ref, dinv_ref, b1_ref, w2_ref, g2_ref):
    dinv = dinv_ref[...]
    x1 = jnp.maximum(
        dinv * (a_ref[0] + a_ref[1] - g1_ref[...]) + b1_ref[...], 0.0)
    g2_ref[...] = dinv * jnp.dot(
        x1, w2_ref[...], preferred_element_type=jnp.float32)


_tc_mid = pl.pallas_call(
    _tc_mid_body,
    grid=(NP // BLK,),
    in_specs=[
        pl.BlockSpec((NC, BLK, NHID), lambda i: (0, i, 0)),
        pl.BlockSpec((BLK, NHID), lambda i: (i, 0)),
        pl.BlockSpec((BLK, 1), lambda i: (i, 0)),
        pl.BlockSpec((1, NHID), lambda i: (0, 0)),
        pl.BlockSpec((NHID, D2P), lambda i: (0, 0)),
    ],
    out_specs=pl.BlockSpec((BLK, D2P), lambda i: (i, 0)),
    out_shape=jax.ShapeDtypeStruct((NP, D2P), jnp.float32),
)


def _tc_post_body(a_ref, g2_ref, dinv_ref, b2_ref, o_ref):
    dinv = dinv_ref[...]
    o_ref[...] = dinv * (a_ref[0] + a_ref[1] - g2_ref[...]) + b2_ref[...]


_tc_post = pl.pallas_call(
    _tc_post_body,
    grid=(NP // BLK,),
    in_specs=[
        pl.BlockSpec((NC, BLK, D2P), lambda i: (0, i, 0)),
        pl.BlockSpec((BLK, D2P), lambda i: (i, 0)),
        pl.BlockSpec((BLK, 1), lambda i: (i, 0)),
        pl.BlockSpec((1, D2P), lambda i: (0, 0)),
    ],
    out_specs=pl.BlockSpec((BLK, D2P), lambda i: (i, 0)),
    out_shape=jax.ShapeDtypeStruct((NP, D2P), jnp.float32),
)


def kernel(x, adj, W1, b1, W2, b2):
    row = adj[0].astype(jnp.int32)
    col = adj[1].astype(jnp.int32)
    pad = E_PAD - E
    rowp = jnp.concatenate(
        [row, jnp.zeros((pad,), jnp.int32)]).reshape(E_PAD // 128, 128)
    colp = jnp.concatenate(
        [col, jnp.full((pad,), N, jnp.int32)]).reshape(E_PAD // 128, 128)
    xp = jnp.pad(x, ((0, NP - N), (0, 0)))

    degp = _sc_degree(colp)
    g1, dinv = _tc_pre(degp[0], degp[1], xp, W1)
    acc1 = _prop128(g1, rowp, colp)

    W2p = jnp.pad(W2, ((0, 0), (0, D2P - NCLASS)))
    b1r = b1.reshape(1, NHID)
    b2p = jnp.pad(b2, (0, D2P - NCLASS)).reshape(1, D2P)

    g2 = _tc_mid(acc1, g1, dinv, b1r, W2p)
    acc2 = _prop48(g2, rowp, colp)
    out = _tc_post(acc2, g2, dinv, b2p)
    return out[:N, :NCLASS]


# speedup vs baseline: 1.6353x; 1.1805x over previous
import functools

import jax
import jax.numpy as jnp
from jax import lax
from jax.experimental import pallas as pl
from jax.experimental.pallas import tpu as pltpu
from jax.experimental.pallas import tpu_sc as plsc

N = 10000
NFEAT = 128
NHID = 128
NCLASS = 40
E = 320000

NC = 2
NS = 16
L = 16
NW = NC * NS

IDX_ROWS_PER_TILE = 80
E_PAD = NW * IDX_ROWS_PER_TILE * 128
GROUPS = 5
G = IDX_ROWS_PER_TILE // GROUPS
NP = 10240
RPT = NP // NS
D2P = 48

_mesh = plsc.VectorSubcoreMesh(core_axis_name="c", subcore_axis_name="s")


@functools.partial(
    pl.kernel,
    out_type=jax.ShapeDtypeStruct((NC, NP, L), jnp.float32),
    mesh=_mesh,
    compiler_params=pltpu.CompilerParams(use_tc_tiling_on_sc=False),
    scratch_types=[
        pltpu.VMEM((G, 128), jnp.int32),
        pltpu.VMEM((128, L), jnp.float32),
        pltpu.VMEM_SHARED((NP, L), jnp.float32),
    ],
)
def _sc_degree(col_hbm, out_hbm, cidx, buf, acc):
    c = lax.axis_index("c")
    s = lax.axis_index("s")
    wid = c * NS + s

    @pl.loop(0, 128)
    def _(i):
        buf[i, :] = jnp.zeros((L,), jnp.float32)

    @pl.loop(0, RPT // 128)
    def _(z):
        pltpu.sync_copy(buf, acc.at[pl.ds(s * RPT + z * 128, 128)])

    @pl.loop(0, 128)
    def _(i):
        buf[i, :] = jnp.full((L,), 1.0, jnp.float32)

    plsc.subcore_barrier()

    @pl.loop(0, GROUPS)
    def _(t):
        pltpu.sync_copy(
            col_hbm.at[pl.ds(wid * IDX_ROWS_PER_TILE + t * G, G)], cidx)
        for jj in range(G):
            pltpu.sync_copy(buf, acc.at[cidx.at[jj]], add=True)

    plsc.subcore_barrier()
    pltpu.sync_copy(acc.at[pl.ds(s * RPT, RPT)],
                    out_hbm.at[c, pl.ds(s * RPT, RPT)])


def _make_prop(D):
    T = IDX_ROWS_PER_TILE
    H = T // 2

    @functools.partial(
        pl.kernel,
        out_type=jax.ShapeDtypeStruct((NC, NP, D), jnp.float32),
        mesh=_mesh,
        compiler_params=pltpu.CompilerParams(
            use_tc_tiling_on_sc=(D % 128 == 0)),
        scratch_types=[
            pltpu.VMEM((H, 128), jnp.int32),
            pltpu.VMEM((H, 128), jnp.int32),
            pltpu.VMEM((128, D), jnp.float32),
            pltpu.VMEM((128, D), jnp.float32),
            pltpu.SemaphoreType.DMA,
            pltpu.SemaphoreType.DMA,
            pltpu.VMEM_SHARED((NP, D), jnp.float32),
        ],
    )
    def _prop(g_hbm, row_hbm, col_hbm, out_hbm, ridx, cidx, bufa, bufb,
              sema, semb, acc):
        c = lax.axis_index("c")
        s = lax.axis_index("s")
        wid = c * NS + s

        pltpu.sync_copy(g_hbm.at[pl.ds(s * RPT, RPT)],
                        acc.at[pl.ds(s * RPT, RPT)])
        plsc.subcore_barrier()

        for h in range(2):
            base = wid * T + h * H
            pltpu.sync_copy(row_hbm.at[pl.ds(base, H)], ridx)
            pltpu.sync_copy(col_hbm.at[pl.ds(base, H)], cidx)
            pltpu.async_copy(g_hbm.at[ridx.at[0]], bufa, sema)

            @pl.loop(0, H, step=2)
            def _(t):
                pltpu.async_copy(g_hbm.at[ridx.at[t + 1]], bufb, semb)
                pltpu.make_async_copy(g_hbm.at[ridx.at[t]], bufa, sema).wait()
                pltpu.sync_copy(bufa, acc.at[cidx.at[t]], add=True)
                tn = jnp.minimum(t + 2, H - 1)
                pltpu.async_copy(g_hbm.at[ridx.at[tn]], bufa, sema)
                pltpu.make_async_copy(
                    g_hbm.at[ridx.at[t + 1]], bufb, semb).wait()
                pltpu.sync_copy(bufb, acc.at[cidx.at[t + 1]], add=True)

            pltpu.make_async_copy(g_hbm.at[ridx.at[H - 1]], bufa, sema).wait()

        plsc.subcore_barrier()
        pltpu.sync_copy(acc.at[pl.ds(s * RPT, RPT)],
                        out_hbm.at[c, pl.ds(s * RPT, RPT)])

    return _prop


def _make_prop_local(D):
    T = IDX_ROWS_PER_TILE
    H = T // 2

    @functools.partial(
        pl.kernel,
        out_type=jax.ShapeDtypeStruct((NC, NP, D), jnp.float32),
        mesh=_mesh,
        compiler_params=pltpu.CompilerParams(
            use_tc_tiling_on_sc=(D % 128 == 0)),
        scratch_types=[
            pltpu.VMEM((H, 128), jnp.int32),
            pltpu.VMEM((H, 128), jnp.int32),
            pltpu.VMEM((128, D), jnp.float32),
            pltpu.VMEM((128, D), jnp.float32),
            pltpu.SemaphoreType.DMA,
            pltpu.SemaphoreType.DMA,
            pltpu.VMEM_SHARED((NP, D), jnp.float32),
            pltpu.VMEM_SHARED((NP, D), jnp.float32),
        ],
    )
    def _prop(g_hbm, row_hbm, col_hbm, out_hbm, ridx, cidx, bufa, bufb,
              sema, semb, gsp, acc):
        c = lax.axis_index("c")
        s = lax.axis_index("s")
        wid = c * NS + s

        pltpu.sync_copy(g_hbm.at[pl.ds(s * RPT, RPT)],
                        gsp.at[pl.ds(s * RPT, RPT)])
        pltpu.sync_copy(g_hbm.at[pl.ds(s * RPT, RPT)],
                        acc.at[pl.ds(s * RPT, RPT)])
        plsc.subcore_barrier()

        for h in range(2):
            base = wid * T + h * H
            pltpu.sync_copy(row_hbm.at[pl.ds(base, H)], ridx)
            pltpu.sync_copy(col_hbm.at[pl.ds(base, H)], cidx)
            pltpu.async_copy(gsp.at[ridx.at[0]], bufa, sema)

            @pl.loop(0, H, step=2)
            def _(t):
                pltpu.async_copy(gsp.at[ridx.at[t + 1]], bufb, semb)
                pltpu.make_async_copy(gsp.at[ridx.at[t]], bufa, sema).wait()
                pltpu.sync_copy(bufa, acc.at[cidx.at[t]], add=True)
                tn = jnp.minimum(t + 2, H - 1)
                pltpu.async_copy(gsp.at[ridx.at[tn]], bufa, sema)
                pltpu.make_async_copy(
                    gsp.at[ridx.at[t + 1]], bufb, semb).wait()
                pltpu.sync_copy(bufb, acc.at[cidx.at[t + 1]], add=True)

            pltpu.make_async_copy(gsp.at[ridx.at[H - 1]], bufa, sema).wait()

        plsc.subcore_barrier()
        pltpu.sync_copy(acc.at[pl.ds(s * RPT, RPT)],
                        out_hbm.at[c, pl.ds(s * RPT, RPT)])

    return _prop


_prop128 = _make_prop(NHID)
_prop48 = _make_prop_local(D2P)

BLK = 1024


def _tc_pre_body(d0_ref, d1_ref, x_ref, w_ref, g_ref, dinv_ref):
    deg = d0_ref[...][:, 0:1] + d1_ref[...][:, 0:1] + 1.0
    dinv = lax.rsqrt(deg)
    h = jnp.dot(x_ref[...], w_ref[...], preferred_element_type=jnp.float32)
    g_ref[...] = h * dinv
    dinv_ref[...] = dinv


_tc_pre = pl.pallas_call(
    _tc_pre_body,
    grid=(NP // BLK,),
    in_specs=[
        pl.BlockSpec((BLK, L), lambda i: (i, 0)),
        pl.BlockSpec((BLK, L), lambda i: (i, 0)),
        pl.BlockSpec((BLK, NFEAT), lambda i: (i, 0)),
        pl.BlockSpec((NFEAT, NHID), lambda i: (0, 0)),
    ],
    out_specs=[
        pl.BlockSpec((BLK, NHID), lambda i: (i, 0)),
        pl.BlockSpec((BLK, 1), lambda i: (i, 0)),
    ],
    out_shape=[
        jax.ShapeDtypeStruct((NP, NHID), jnp.float32),
        jax.ShapeDtypeStruct((NP, 1), jnp.float32),
    ],
)


def _tc_mid_body(a_ref, g1_ref, dinv_ref, b1_ref, w2_ref, g2_ref):
    dinv = dinv_ref[...]
    x1 = jnp.maximum(
        dinv * (a_ref[0] + a_ref[1] - g1_ref[...]) + b1_ref[...], 0.0)
    g2_ref[...] = dinv * jnp.dot(
        x1, w2_ref[...], preferred_element_type=jnp.float32)


_tc_mid = pl.pallas_call(
    _tc_mid_body,
    grid=(NP // BLK,),
    in_specs=[
        pl.BlockSpec((NC, BLK, NHID), lambda i: (0, i, 0)),
        pl.BlockSpec((BLK, NHID), lambda i: (i, 0)),
        pl.BlockSpec((BLK, 1), lambda i: (i, 0)),
        pl.BlockSpec((1, NHID), lambda i: (0, 0)),
        pl.BlockSpec((NHID, D2P), lambda i: (0, 0)),
    ],
    out_specs=pl.BlockSpec((BLK, D2P), lambda i: (i, 0)),
    out_shape=jax.ShapeDtypeStruct((NP, D2P), jnp.float32),
)


def _tc_post_body(a_ref, g2_ref, dinv_ref, b2_ref, o_ref):
    dinv = dinv_ref[...]
    o_ref[...] = dinv * (a_ref[0] + a_ref[1] - g2_ref[...]) + b2_ref[...]


_tc_post = pl.pallas_call(
    _tc_post_body,
    grid=(NP // BLK,),
    in_specs=[
        pl.BlockSpec((NC, BLK, D2P), lambda i: (0, i, 0)),
        pl.BlockSpec((BLK, D2P), lambda i: (i, 0)),
        pl.BlockSpec((BLK, 1), lambda i: (i, 0)),
        pl.BlockSpec((1, D2P), lambda i: (0, 0)),
    ],
    out_specs=pl.BlockSpec((BLK, D2P), lambda i: (i, 0)),
    out_shape=jax.ShapeDtypeStruct((NP, D2P), jnp.float32),
)


def kernel(x, adj, W1, b1, W2, b2):
    row = adj[0].astype(jnp.int32)
    col = adj[1].astype(jnp.int32)
    pad = E_PAD - E
    rowp = jnp.concatenate(
        [row, jnp.zeros((pad,), jnp.int32)]).reshape(E_PAD // 128, 128)
    colp = jnp.concatenate(
        [col, jnp.full((pad,), N, jnp.int32)]).reshape(E_PAD // 128, 128)
    xp = jnp.pad(x, ((0, NP - N), (0, 0)))

    degp = _sc_degree(colp)
    g1, dinv = _tc_pre(degp[0], degp[1], xp, W1)
    acc1 = _prop128(g1, rowp, colp)

    W2p = jnp.pad(W2, ((0, 0), (0, D2P - NCLASS)))
    b1r = b1.reshape(1, NHID)
    b2p = jnp.pad(b2, (0, D2P - NCLASS)).reshape(1, D2P)

    g2 = _tc_mid(acc1, g1, dinv, b1r, W2p)
    acc2 = _prop48(g2, rowp, colp)
    out = _tc_post(acc2, g2, dinv, b2p)
    return out[:N, :NCLASS]


# speedup vs baseline: 2.6030x; 1.5917x over previous
import functools

import jax
import jax.numpy as jnp
from jax import lax
from jax.experimental import pallas as pl
from jax.experimental.pallas import tpu as pltpu
from jax.experimental.pallas import tpu_sc as plsc

N = 10000
NFEAT = 128
NHID = 128
NCLASS = 40
E = 320000

NC = 2
NS = 16
L = 16
NW = NC * NS

IDX_ROWS_PER_TILE = 80
E_PAD = NW * IDX_ROWS_PER_TILE * 128
GROUPS = 5
G = IDX_ROWS_PER_TILE // GROUPS
NP = 10240
RPT = NP // NS
D2P = 48

_mesh = plsc.VectorSubcoreMesh(core_axis_name="c", subcore_axis_name="s")


@functools.partial(
    pl.kernel,
    out_type=jax.ShapeDtypeStruct((NC, NP, L), jnp.float32),
    mesh=_mesh,
    compiler_params=pltpu.CompilerParams(use_tc_tiling_on_sc=False),
    scratch_types=[
        pltpu.VMEM((G, 128), jnp.int32),
        pltpu.VMEM((128, L), jnp.float32),
        pltpu.VMEM_SHARED((NP, L), jnp.float32),
    ],
)
def _sc_degree(col_hbm, out_hbm, cidx, buf, acc):
    c = lax.axis_index("c")
    s = lax.axis_index("s")
    wid = c * NS + s

    @pl.loop(0, 128)
    def _(i):
        buf[i, :] = jnp.zeros((L,), jnp.float32)

    @pl.loop(0, RPT // 128)
    def _(z):
        pltpu.sync_copy(buf, acc.at[pl.ds(s * RPT + z * 128, 128)])

    @pl.loop(0, 128)
    def _(i):
        buf[i, :] = jnp.full((L,), 1.0, jnp.float32)

    plsc.subcore_barrier()

    @pl.loop(0, GROUPS)
    def _(t):
        pltpu.sync_copy(
            col_hbm.at[pl.ds(wid * IDX_ROWS_PER_TILE + t * G, G)], cidx)
        for jj in range(G):
            pltpu.sync_copy(buf, acc.at[cidx.at[jj]], add=True)

    plsc.subcore_barrier()
    pltpu.sync_copy(acc.at[pl.ds(s * RPT, RPT)],
                    out_hbm.at[c, pl.ds(s * RPT, RPT)])


def _make_prop(D):
    T = IDX_ROWS_PER_TILE
    H = T // 2

    @functools.partial(
        pl.kernel,
        out_type=jax.ShapeDtypeStruct((NC, NP, D), jnp.float32),
        mesh=_mesh,
        compiler_params=pltpu.CompilerParams(
            use_tc_tiling_on_sc=(D % 128 == 0)),
        scratch_types=[
            pltpu.VMEM((H, 128), jnp.int32),
            pltpu.VMEM((H, 128), jnp.int32),
            pltpu.VMEM((128, D), jnp.float32),
            pltpu.VMEM((128, D), jnp.float32),
            pltpu.SemaphoreType.DMA,
            pltpu.SemaphoreType.DMA,
            pltpu.VMEM_SHARED((NP, D), jnp.float32),
        ],
    )
    def _prop(g_hbm, row_hbm, col_hbm, out_hbm, ridx, cidx, bufa, bufb,
              sema, semb, acc):
        c = lax.axis_index("c")
        s = lax.axis_index("s")
        wid = c * NS + s

        pltpu.sync_copy(g_hbm.at[pl.ds(s * RPT, RPT)],
                        acc.at[pl.ds(s * RPT, RPT)])
        plsc.subcore_barrier()

        for h in range(2):
            base = wid * T + h * H
            pltpu.sync_copy(row_hbm.at[pl.ds(base, H)], ridx)
            pltpu.sync_copy(col_hbm.at[pl.ds(base, H)], cidx)
            pltpu.async_copy(g_hbm.at[ridx.at[0]], bufa, sema)

            @pl.loop(0, H, step=2)
            def _(t):
                pltpu.async_copy(g_hbm.at[ridx.at[t + 1]], bufb, semb)
                pltpu.make_async_copy(g_hbm.at[ridx.at[t]], bufa, sema).wait()
                pltpu.sync_copy(bufa, acc.at[cidx.at[t]], add=True)
                tn = jnp.minimum(t + 2, H - 1)
                pltpu.async_copy(g_hbm.at[ridx.at[tn]], bufa, sema)
                pltpu.make_async_copy(
                    g_hbm.at[ridx.at[t + 1]], bufb, semb).wait()
                pltpu.sync_copy(bufb, acc.at[cidx.at[t + 1]], add=True)

            pltpu.make_async_copy(g_hbm.at[ridx.at[H - 1]], bufa, sema).wait()

        plsc.subcore_barrier()
        pltpu.sync_copy(acc.at[pl.ds(s * RPT, RPT)],
                        out_hbm.at[c, pl.ds(s * RPT, RPT)])

    return _prop


def _make_prop_local2(D):
    T = IDX_ROWS_PER_TILE
    H = T // 2

    @functools.partial(
        pl.kernel,
        out_type=jax.ShapeDtypeStruct((NC, 2, NP, D), jnp.float32),
        mesh=_mesh,
        compiler_params=pltpu.CompilerParams(use_tc_tiling_on_sc=False),
        scratch_types=[
            pltpu.VMEM((H, 128), jnp.int32),
            pltpu.VMEM((H, 128), jnp.int32),
            pltpu.VMEM((128, D), jnp.float32),
            pltpu.VMEM((128, D), jnp.float32),
            pltpu.SemaphoreType.DMA,
            pltpu.SemaphoreType.DMA,
            pltpu.VMEM_SHARED((NP, D), jnp.float32),
            pltpu.VMEM_SHARED((NP, D), jnp.float32),
        ],
    )
    def _prop(g_hbm, row_hbm, col_hbm, out_hbm, ridx, cidx, bufa, bufb,
              sema, semb, gsp, acc):
        c = lax.axis_index("c")
        s = lax.axis_index("s")
        wid = c * NS + s

        for p in range(2):
            pltpu.sync_copy(g_hbm.at[p, pl.ds(s * RPT, RPT)],
                            gsp.at[pl.ds(s * RPT, RPT)])
            pltpu.sync_copy(g_hbm.at[p, pl.ds(s * RPT, RPT)],
                            acc.at[pl.ds(s * RPT, RPT)])
            plsc.subcore_barrier()

            for h in range(2):
                base = wid * T + h * H
                pltpu.sync_copy(row_hbm.at[pl.ds(base, H)], ridx)
                pltpu.sync_copy(col_hbm.at[pl.ds(base, H)], cidx)
                pltpu.async_copy(gsp.at[ridx.at[0]], bufa, sema)

                @pl.loop(0, H, step=2)
                def _(t):
                    pltpu.async_copy(gsp.at[ridx.at[t + 1]], bufb, semb)
                    pltpu.make_async_copy(
                        gsp.at[ridx.at[t]], bufa, sema).wait()
                    pltpu.sync_copy(bufa, acc.at[cidx.at[t]], add=True)
                    tn = jnp.minimum(t + 2, H - 1)
                    pltpu.async_copy(gsp.at[ridx.at[tn]], bufa, sema)
                    pltpu.make_async_copy(
                        gsp.at[ridx.at[t + 1]], bufb, semb).wait()
                    pltpu.sync_copy(bufb, acc.at[cidx.at[t + 1]], add=True)

                pltpu.make_async_copy(
                    gsp.at[ridx.at[H - 1]], bufa, sema).wait()

            plsc.subcore_barrier()
            pltpu.sync_copy(acc.at[pl.ds(s * RPT, RPT)],
                            out_hbm.at[c, p, pl.ds(s * RPT, RPT)])
            if p == 0:
                plsc.subcore_barrier()

    return _prop


def _make_prop_local(D):
    T = IDX_ROWS_PER_TILE
    H = T // 2

    @functools.partial(
        pl.kernel,
        out_type=jax.ShapeDtypeStruct((NC, NP, D), jnp.float32),
        mesh=_mesh,
        compiler_params=pltpu.CompilerParams(
            use_tc_tiling_on_sc=(D % 128 == 0)),
        scratch_types=[
            pltpu.VMEM((H, 128), jnp.int32),
            pltpu.VMEM((H, 128), jnp.int32),
            pltpu.VMEM((128, D), jnp.float32),
            pltpu.VMEM((128, D), jnp.float32),
            pltpu.SemaphoreType.DMA,
            pltpu.SemaphoreType.DMA,
            pltpu.VMEM_SHARED((NP, D), jnp.float32),
            pltpu.VMEM_SHARED((NP, D), jnp.float32),
        ],
    )
    def _prop(g_hbm, row_hbm, col_hbm, out_hbm, ridx, cidx, bufa, bufb,
              sema, semb, gsp, acc):
        c = lax.axis_index("c")
        s = lax.axis_index("s")
        wid = c * NS + s

        pltpu.sync_copy(g_hbm.at[pl.ds(s * RPT, RPT)],
                        gsp.at[pl.ds(s * RPT, RPT)])
        pltpu.sync_copy(g_hbm.at[pl.ds(s * RPT, RPT)],
                        acc.at[pl.ds(s * RPT, RPT)])
        plsc.subcore_barrier()

        for h in range(2):
            base = wid * T + h * H
            pltpu.sync_copy(row_hbm.at[pl.ds(base, H)], ridx)
            pltpu.sync_copy(col_hbm.at[pl.ds(base, H)], cidx)
            pltpu.async_copy(gsp.at[ridx.at[0]], bufa, sema)

            @pl.loop(0, H, step=2)
            def _(t):
                pltpu.async_copy(gsp.at[ridx.at[t + 1]], bufb, semb)
                pltpu.make_async_copy(gsp.at[ridx.at[t]], bufa, sema).wait()
                pltpu.sync_copy(bufa, acc.at[cidx.at[t]], add=True)
                tn = jnp.minimum(t + 2, H - 1)
                pltpu.async_copy(gsp.at[ridx.at[tn]], bufa, sema)
                pltpu.make_async_copy(
                    gsp.at[ridx.at[t + 1]], bufb, semb).wait()
                pltpu.sync_copy(bufb, acc.at[cidx.at[t + 1]], add=True)

            pltpu.make_async_copy(gsp.at[ridx.at[H - 1]], bufa, sema).wait()

        plsc.subcore_barrier()
        pltpu.sync_copy(acc.at[pl.ds(s * RPT, RPT)],
                        out_hbm.at[c, pl.ds(s * RPT, RPT)])

    return _prop


_prop128 = _make_prop_local2(NHID // 2)
_prop48 = _make_prop_local(D2P)

BLK = 1024


HH = NHID // 2


def _tc_pre_body(d0_ref, d1_ref, x_ref, w_ref, g_ref, dinv_ref):
    deg = d0_ref[...][:, 0:1] + d1_ref[...][:, 0:1] + 1.0
    dinv = lax.rsqrt(deg)
    h = jnp.dot(x_ref[...], w_ref[...], preferred_element_type=jnp.float32)
    g = h * dinv
    g_ref[0] = g[:, :HH]
    g_ref[1] = g[:, HH:]
    dinv_ref[...] = dinv


_tc_pre = pl.pallas_call(
    _tc_pre_body,
    grid=(NP // BLK,),
    in_specs=[
        pl.BlockSpec((BLK, L), lambda i: (i, 0)),
        pl.BlockSpec((BLK, L), lambda i: (i, 0)),
        pl.BlockSpec((BLK, NFEAT), lambda i: (i, 0)),
        pl.BlockSpec((NFEAT, NHID), lambda i: (0, 0)),
    ],
    out_specs=[
        pl.BlockSpec((2, BLK, HH), lambda i: (0, i, 0)),
        pl.BlockSpec((BLK, 1), lambda i: (i, 0)),
    ],
    out_shape=[
        jax.ShapeDtypeStruct((2, NP, HH), jnp.float32),
        jax.ShapeDtypeStruct((NP, 1), jnp.float32),
    ],
)


def _tc_mid_body(a_ref, g1_ref, dinv_ref, b1_ref, w2_ref, g2_ref):
    dinv = dinv_ref[...]
    acc = jnp.concatenate(
        [a_ref[0, 0] + a_ref[1, 0], a_ref[0, 1] + a_ref[1, 1]], axis=1)
    g1 = jnp.concatenate([g1_ref[0], g1_ref[1]], axis=1)
    x1 = jnp.maximum(dinv * (acc - g1) + b1_ref[...], 0.0)
    g2_ref[...] = dinv * jnp.dot(
        x1, w2_ref[...], preferred_element_type=jnp.float32)


_tc_mid = pl.pallas_call(
    _tc_mid_body,
    grid=(NP // BLK,),
    in_specs=[
        pl.BlockSpec((NC, 2, BLK, HH), lambda i: (0, 0, i, 0)),
        pl.BlockSpec((2, BLK, HH), lambda i: (0, i, 0)),
        pl.BlockSpec((BLK, 1), lambda i: (i, 0)),
        pl.BlockSpec((1, NHID), lambda i: (0, 0)),
        pl.BlockSpec((NHID, D2P), lambda i: (0, 0)),
    ],
    out_specs=pl.BlockSpec((BLK, D2P), lambda i: (i, 0)),
    out_shape=jax.ShapeDtypeStruct((NP, D2P), jnp.float32),
)


def _tc_post_body(a_ref, g2_ref, dinv_ref, b2_ref, o_ref):
    dinv = dinv_ref[...]
    o_ref[...] = dinv * (a_ref[0] + a_ref[1] - g2_ref[...]) + b2_ref[...]


_tc_post = pl.pallas_call(
    _tc_post_body,
    grid=(NP // BLK,),
    in_specs=[
        pl.BlockSpec((NC, BLK, D2P), lambda i: (0, i, 0)),
        pl.BlockSpec((BLK, D2P), lambda i: (i, 0)),
        pl.BlockSpec((BLK, 1), lambda i: (i, 0)),
        pl.BlockSpec((1, D2P), lambda i: (0, 0)),
    ],
    out_specs=pl.BlockSpec((BLK, D2P), lambda i: (i, 0)),
    out_shape=jax.ShapeDtypeStruct((NP, D2P), jnp.float32),
)


def kernel(x, adj, W1, b1, W2, b2):
    row = adj[0].astype(jnp.int32)
    col = adj[1].astype(jnp.int32)
    pad = E_PAD - E
    rowp = jnp.concatenate(
        [row, jnp.zeros((pad,), jnp.int32)]).reshape(E_PAD // 128, 128)
    colp = jnp.concatenate(
        [col, jnp.full((pad,), N, jnp.int32)]).reshape(E_PAD // 128, 128)
    xp = jnp.pad(x, ((0, NP - N), (0, 0)))

    degp = _sc_degree(colp)
    g1, dinv = _tc_pre(degp[0], degp[1], xp, W1)
    acc1 = _prop128(g1, rowp, colp)

    W2p = jnp.pad(W2, ((0, 0), (0, D2P - NCLASS)))
    b1r = b1.reshape(1, NHID)
    b2p = jnp.pad(b2, (0, D2P - NCLASS)).reshape(1, D2P)

    g2 = _tc_mid(acc1, g1, dinv, b1r, W2p)
    acc2 = _prop48(g2, rowp, colp)
    out = _tc_post(acc2, g2, dinv, b2p)
    return out[:N, :NCLASS]


# speedup vs baseline: 2.6994x; 1.0370x over previous
import functools

import jax
import jax.numpy as jnp
from jax import lax
from jax.experimental import pallas as pl
from jax.experimental.pallas import tpu as pltpu
from jax.experimental.pallas import tpu_sc as plsc

N = 10000
NFEAT = 128
NHID = 128
NCLASS = 40
E = 320000

NC = 2
NS = 16
L = 16
NW = NC * NS

IDX_ROWS_PER_TILE = 80
E_PAD = NW * IDX_ROWS_PER_TILE * 128
GROUPS = 5
G = IDX_ROWS_PER_TILE // GROUPS
NP = 10240
RPT = NP // NS
D2P = 48

_mesh = plsc.VectorSubcoreMesh(core_axis_name="c", subcore_axis_name="s")


@functools.partial(
    pl.kernel,
    out_type=jax.ShapeDtypeStruct((NC, NP, L), jnp.float32),
    mesh=_mesh,
    scratch_types=[
        pltpu.VMEM((IDX_ROWS_PER_TILE, 128), jnp.int32),
        pltpu.VMEM((128, L), jnp.float32),
        pltpu.SemaphoreType.DMA,
        pltpu.VMEM_SHARED((NP, L), jnp.float32),
    ],
)
def _sc_degree(col_hbm, out_hbm, cidx, buf, sem, acc):
    c = lax.axis_index("c")
    s = lax.axis_index("s")
    wid = c * NS + s
    T = IDX_ROWS_PER_TILE

    @pl.loop(0, 128)
    def _(i):
        buf[i, :] = jnp.zeros((L,), jnp.float32)

    @pl.loop(0, RPT // 128)
    def _(z):
        pltpu.sync_copy(buf, acc.at[pl.ds(s * RPT + z * 128, 128)])

    @pl.loop(0, 128)
    def _(i):
        buf[i, :] = jnp.full((L,), 1.0, jnp.float32)

    pltpu.sync_copy(col_hbm.at[pl.ds(wid * T, T)], cidx)
    plsc.subcore_barrier()

    @pl.loop(0, T)
    def _(t):
        pltpu.async_copy(buf, acc.at[cidx.at[t]], sem, add=True)

    @pl.loop(0, T)
    def _(t):
        pltpu.make_async_copy(buf, acc.at[cidx.at[t]], sem).wait()

    plsc.subcore_barrier()
    pltpu.sync_copy(acc.at[pl.ds(s * RPT, RPT)],
                    out_hbm.at[c, pl.ds(s * RPT, RPT)])


def _make_prop(D):
    T = IDX_ROWS_PER_TILE
    H = T // 2

    @functools.partial(
        pl.kernel,
        out_type=jax.ShapeDtypeStruct((NC, NP, D), jnp.float32),
        mesh=_mesh,
        compiler_params=pltpu.CompilerParams(
            use_tc_tiling_on_sc=(D % 128 == 0)),
        scratch_types=[
            pltpu.VMEM((H, 128), jnp.int32),
            pltpu.VMEM((H, 128), jnp.int32),
            pltpu.VMEM((128, D), jnp.float32),
            pltpu.VMEM((128, D), jnp.float32),
            pltpu.SemaphoreType.DMA,
            pltpu.SemaphoreType.DMA,
            pltpu.VMEM_SHARED((NP, D), jnp.float32),
        ],
    )
    def _prop(g_hbm, row_hbm, col_hbm, out_hbm, ridx, cidx, bufa, bufb,
              sema, semb, acc):
        c = lax.axis_index("c")
        s = lax.axis_index("s")
        wid = c * NS + s

        pltpu.sync_copy(g_hbm.at[pl.ds(s * RPT, RPT)],
                        acc.at[pl.ds(s * RPT, RPT)])
        plsc.subcore_barrier()

        for h in range(2):
            base = wid * T + h * H
            pltpu.sync_copy(row_hbm.at[pl.ds(base, H)], ridx)
            pltpu.sync_copy(col_hbm.at[pl.ds(base, H)], cidx)
            pltpu.async_copy(g_hbm.at[ridx.at[0]], bufa, sema)

            @pl.loop(0, H, step=2)
            def _(t):
                pltpu.async_copy(g_hbm.at[ridx.at[t + 1]], bufb, semb)
                pltpu.make_async_copy(g_hbm.at[ridx.at[t]], bufa, sema).wait()
                pltpu.sync_copy(bufa, acc.at[cidx.at[t]], add=True)
                tn = jnp.minimum(t + 2, H - 1)
                pltpu.async_copy(g_hbm.at[ridx.at[tn]], bufa, sema)
                pltpu.make_async_copy(
                    g_hbm.at[ridx.at[t + 1]], bufb, semb).wait()
                pltpu.sync_copy(bufb, acc.at[cidx.at[t + 1]], add=True)

            pltpu.make_async_copy(g_hbm.at[ridx.at[H - 1]], bufa, sema).wait()

        plsc.subcore_barrier()
        pltpu.sync_copy(acc.at[pl.ds(s * RPT, RPT)],
                        out_hbm.at[c, pl.ds(s * RPT, RPT)])

    return _prop


def _edge_pipeline(gsp, acc, ridx, cidx, bufa, bufb, sga, sgb, ssa, ssb):
    T = IDX_ROWS_PER_TILE
    pltpu.async_copy(gsp.at[ridx.at[0]], bufa, sga)
    pltpu.make_async_copy(gsp.at[ridx.at[0]], bufa, sga).wait()
    pltpu.async_copy(bufa, acc.at[cidx.at[0]], ssa, add=True)
    pltpu.async_copy(gsp.at[ridx.at[1]], bufb, sgb)

    @pl.loop(1, T - 1, step=2)
    def _(t):
        pltpu.make_async_copy(gsp.at[ridx.at[t]], bufb, sgb).wait()
        pltpu.async_copy(bufb, acc.at[cidx.at[t]], ssb, add=True)
        pltpu.make_async_copy(bufa, acc.at[cidx.at[t - 1]], ssa).wait()
        pltpu.async_copy(gsp.at[ridx.at[t + 1]], bufa, sga)
        pltpu.make_async_copy(gsp.at[ridx.at[t + 1]], bufa, sga).wait()
        pltpu.async_copy(bufa, acc.at[cidx.at[t + 1]], ssa, add=True)
        pltpu.make_async_copy(bufb, acc.at[cidx.at[t]], ssb).wait()
        pltpu.async_copy(gsp.at[ridx.at[t + 2]], bufb, sgb)

    pltpu.make_async_copy(gsp.at[ridx.at[T - 1]], bufb, sgb).wait()
    pltpu.async_copy(bufb, acc.at[cidx.at[T - 1]], ssb, add=True)
    pltpu.make_async_copy(bufa, acc.at[cidx.at[T - 2]], ssa).wait()
    pltpu.make_async_copy(bufb, acc.at[cidx.at[T - 1]], ssb).wait()


def _prop_scratch(D):
    return [
        pltpu.VMEM((IDX_ROWS_PER_TILE, 128), jnp.int32),
        pltpu.VMEM((IDX_ROWS_PER_TILE, 128), jnp.int32),
        pltpu.VMEM((128, D), jnp.float32),
        pltpu.VMEM((128, D), jnp.float32),
        pltpu.SemaphoreType.DMA,
        pltpu.SemaphoreType.DMA,
        pltpu.SemaphoreType.DMA,
        pltpu.SemaphoreType.DMA,
        pltpu.VMEM_SHARED((NP, D), jnp.float32),
        pltpu.VMEM_SHARED((NP, D), jnp.float32),
    ]


def _make_prop_local2(D):
    T = IDX_ROWS_PER_TILE

    @functools.partial(
        pl.kernel,
        out_type=jax.ShapeDtypeStruct((NC, 2, NP, D), jnp.float32),
        mesh=_mesh,
        compiler_params=pltpu.CompilerParams(use_tc_tiling_on_sc=False),
        scratch_types=_prop_scratch(D),
    )
    def _prop(g_hbm, row_hbm, col_hbm, out_hbm, ridx, cidx, bufa, bufb,
              sga, sgb, ssa, ssb, gsp, acc):
        c = lax.axis_index("c")
        s = lax.axis_index("s")
        wid = c * NS + s

        pltpu.sync_copy(row_hbm.at[pl.ds(wid * T, T)], ridx)
        pltpu.sync_copy(col_hbm.at[pl.ds(wid * T, T)], cidx)

        for p in range(2):
            pltpu.sync_copy(g_hbm.at[p, pl.ds(s * RPT, RPT)],
                            gsp.at[pl.ds(s * RPT, RPT)])
            pltpu.sync_copy(g_hbm.at[p, pl.ds(s * RPT, RPT)],
                            acc.at[pl.ds(s * RPT, RPT)])
            plsc.subcore_barrier()

            _edge_pipeline(gsp, acc, ridx, cidx, bufa, bufb,
                           sga, sgb, ssa, ssb)

            plsc.subcore_barrier()
            pltpu.sync_copy(acc.at[pl.ds(s * RPT, RPT)],
                            out_hbm.at[c, p, pl.ds(s * RPT, RPT)])
            if p == 0:
                plsc.subcore_barrier()

    return _prop


def _make_prop_local(D):
    T = IDX_ROWS_PER_TILE

    @functools.partial(
        pl.kernel,
        out_type=jax.ShapeDtypeStruct((NC, NP, D), jnp.float32),
        mesh=_mesh,
        compiler_params=pltpu.CompilerParams(
            use_tc_tiling_on_sc=(D % 128 == 0)),
        scratch_types=_prop_scratch(D),
    )
    def _prop(g_hbm, row_hbm, col_hbm, out_hbm, ridx, cidx, bufa, bufb,
              sga, sgb, ssa, ssb, gsp, acc):
        c = lax.axis_index("c")
        s = lax.axis_index("s")
        wid = c * NS + s

        pltpu.sync_copy(row_hbm.at[pl.ds(wid * T, T)], ridx)
        pltpu.sync_copy(col_hbm.at[pl.ds(wid * T, T)], cidx)
        pltpu.sync_copy(g_hbm.at[pl.ds(s * RPT, RPT)],
                        gsp.at[pl.ds(s * RPT, RPT)])
        pltpu.sync_copy(g_hbm.at[pl.ds(s * RPT, RPT)],
                        acc.at[pl.ds(s * RPT, RPT)])
        plsc.subcore_barrier()

        _edge_pipeline(gsp, acc, ridx, cidx, bufa, bufb, sga, sgb, ssa, ssb)

        plsc.subcore_barrier()
        pltpu.sync_copy(acc.at[pl.ds(s * RPT, RPT)],
                        out_hbm.at[c, pl.ds(s * RPT, RPT)])

    return _prop


_prop128 = _make_prop_local2(NHID // 2)
_prop48 = _make_prop_local(D2P)

BLK = 1024


HH = NHID // 2


def _tc_pre_body(d0_ref, d1_ref, x_ref, w_ref, g_ref, dinv_ref):
    deg = d0_ref[...][:, 0:1] + d1_ref[...][:, 0:1] + 1.0
    dinv = lax.rsqrt(deg)
    h = jnp.dot(x_ref[...], w_ref[...], preferred_element_type=jnp.float32)
    g = h * dinv
    g_ref[0] = g[:, :HH]
    g_ref[1] = g[:, HH:]
    dinv_ref[...] = dinv


_tc_pre = pl.pallas_call(
    _tc_pre_body,
    grid=(NP // BLK,),
    in_specs=[
        pl.BlockSpec((BLK, L), lambda i: (i, 0)),
        pl.BlockSpec((BLK, L), lambda i: (i, 0)),
        pl.BlockSpec((BLK, NFEAT), lambda i: (i, 0)),
        pl.BlockSpec((NFEAT, NHID), lambda i: (0, 0)),
    ],
    out_specs=[
        pl.BlockSpec((2, BLK, HH), lambda i: (0, i, 0)),
        pl.BlockSpec((BLK, 1), lambda i: (i, 0)),
    ],
    out_shape=[
        jax.ShapeDtypeStruct((2, NP, HH), jnp.float32),
        jax.ShapeDtypeStruct((NP, 1), jnp.float32),
    ],
)


def _tc_mid_body(a_ref, g1_ref, dinv_ref, b1_ref, w2_ref, g2_ref):
    dinv = dinv_ref[...]
    acc = jnp.concatenate(
        [a_ref[0, 0] + a_ref[1, 0], a_ref[0, 1] + a_ref[1, 1]], axis=1)
    g1 = jnp.concatenate([g1_ref[0], g1_ref[1]], axis=1)
    x1 = jnp.maximum(dinv * (acc - g1) + b1_ref[...], 0.0)
    g2_ref[...] = dinv * jnp.dot(
        x1, w2_ref[...], preferred_element_type=jnp.float32)


_tc_mid = pl.pallas_call(
    _tc_mid_body,
    grid=(NP // BLK,),
    in_specs=[
        pl.BlockSpec((NC, 2, BLK, HH), lambda i: (0, 0, i, 0)),
        pl.BlockSpec((2, BLK, HH), lambda i: (0, i, 0)),
        pl.BlockSpec((BLK, 1), lambda i: (i, 0)),
        pl.BlockSpec((1, NHID), lambda i: (0, 0)),
        pl.BlockSpec((NHID, D2P), lambda i: (0, 0)),
    ],
    out_specs=pl.BlockSpec((BLK, D2P), lambda i: (i, 0)),
    out_shape=jax.ShapeDtypeStruct((NP, D2P), jnp.float32),
)


def _tc_post_body(a_ref, g2_ref, dinv_ref, b2_ref, o_ref):
    dinv = dinv_ref[...]
    o_ref[...] = dinv * (a_ref[0] + a_ref[1] - g2_ref[...]) + b2_ref[...]


_tc_post = pl.pallas_call(
    _tc_post_body,
    grid=(NP // BLK,),
    in_specs=[
        pl.BlockSpec((NC, BLK, D2P), lambda i: (0, i, 0)),
        pl.BlockSpec((BLK, D2P), lambda i: (i, 0)),
        pl.BlockSpec((BLK, 1), lambda i: (i, 0)),
        pl.BlockSpec((1, D2P), lambda i: (0, 0)),
    ],
    out_specs=pl.BlockSpec((BLK, D2P), lambda i: (i, 0)),
    out_shape=jax.ShapeDtypeStruct((NP, D2P), jnp.float32),
)


def kernel(x, adj, W1, b1, W2, b2):
    row = adj[0].astype(jnp.int32)
    col = adj[1].astype(jnp.int32)
    pad = E_PAD - E
    rowp = jnp.concatenate(
        [row, jnp.zeros((pad,), jnp.int32)]).reshape(E_PAD // 128, 128)
    colp = jnp.concatenate(
        [col, jnp.full((pad,), N, jnp.int32)]).reshape(E_PAD // 128, 128)
    xp = jnp.pad(x, ((0, NP - N), (0, 0)))

    degp = _sc_degree(colp)
    g1, dinv = _tc_pre(degp[0], degp[1], xp, W1)
    acc1 = _prop128(g1, rowp, colp)

    W2p = jnp.pad(W2, ((0, 0), (0, D2P - NCLASS)))
    b1r = b1.reshape(1, NHID)
    b2p = jnp.pad(b2, (0, D2P - NCLASS)).reshape(1, D2P)

    g2 = _tc_mid(acc1, g1, dinv, b1r, W2p)
    acc2 = _prop48(g2, rowp, colp)
    out = _tc_post(acc2, g2, dinv, b2p)
    return out[:N, :NCLASS]


# speedup vs baseline: 2.7043x; 1.0018x over previous
import functools

import jax
import jax.numpy as jnp
from jax import lax
from jax.experimental import pallas as pl
from jax.experimental.pallas import tpu as pltpu
from jax.experimental.pallas import tpu_sc as plsc

N = 10000
NFEAT = 128
NHID = 128
NCLASS = 40
E = 320000

NC = 2
NS = 16
L = 16
NW = NC * NS

IDX_ROWS_PER_TILE = 80
E_PAD = NW * IDX_ROWS_PER_TILE * 128
GROUPS = 5
G = IDX_ROWS_PER_TILE // GROUPS
NP = 10240
RPT = NP // NS
D2P = 48

_mesh = plsc.VectorSubcoreMesh(core_axis_name="c", subcore_axis_name="s")


@functools.partial(
    pl.kernel,
    out_type=jax.ShapeDtypeStruct((NC, NP, L), jnp.float32),
    mesh=_mesh,
    scratch_types=[
        pltpu.VMEM((IDX_ROWS_PER_TILE, 128), jnp.int32),
        pltpu.VMEM((128, L), jnp.float32),
        pltpu.SemaphoreType.DMA,
        pltpu.VMEM_SHARED((NP, L), jnp.float32),
    ],
)
def _sc_degree(col_hbm, out_hbm, cidx, buf, sem, acc):
    c = lax.axis_index("c")
    s = lax.axis_index("s")
    wid = c * NS + s
    T = IDX_ROWS_PER_TILE

    @pl.loop(0, 128)
    def _(i):
        buf[i, :] = jnp.zeros((L,), jnp.float32)

    @pl.loop(0, RPT // 128)
    def _(z):
        pltpu.sync_copy(buf, acc.at[pl.ds(s * RPT + z * 128, 128)])

    @pl.loop(0, 128)
    def _(i):
        buf[i, :] = jnp.full((L,), 1.0, jnp.float32)

    pltpu.sync_copy(col_hbm.at[pl.ds(wid * T, T)], cidx)
    plsc.subcore_barrier()

    @pl.loop(0, T)
    def _(t):
        pltpu.async_copy(buf, acc.at[cidx.at[t]], sem, add=True)

    @pl.loop(0, T)
    def _(t):
        pltpu.make_async_copy(buf, acc.at[cidx.at[t]], sem).wait()

    plsc.subcore_barrier()
    pltpu.sync_copy(acc.at[pl.ds(s * RPT, RPT)],
                    out_hbm.at[c, pl.ds(s * RPT, RPT)])


def _make_prop(D):
    T = IDX_ROWS_PER_TILE
    H = T // 2

    @functools.partial(
        pl.kernel,
        out_type=jax.ShapeDtypeStruct((NC, NP, D), jnp.float32),
        mesh=_mesh,
        compiler_params=pltpu.CompilerParams(
            use_tc_tiling_on_sc=(D % 128 == 0)),
        scratch_types=[
            pltpu.VMEM((H, 128), jnp.int32),
            pltpu.VMEM((H, 128), jnp.int32),
            pltpu.VMEM((128, D), jnp.float32),
            pltpu.VMEM((128, D), jnp.float32),
            pltpu.SemaphoreType.DMA,
            pltpu.SemaphoreType.DMA,
            pltpu.VMEM_SHARED((NP, D), jnp.float32),
        ],
    )
    def _prop(g_hbm, row_hbm, col_hbm, out_hbm, ridx, cidx, bufa, bufb,
              sema, semb, acc):
        c = lax.axis_index("c")
        s = lax.axis_index("s")
        wid = c * NS + s

        pltpu.sync_copy(g_hbm.at[pl.ds(s * RPT, RPT)],
                        acc.at[pl.ds(s * RPT, RPT)])
        plsc.subcore_barrier()

        for h in range(2):
            base = wid * T + h * H
            pltpu.sync_copy(row_hbm.at[pl.ds(base, H)], ridx)
            pltpu.sync_copy(col_hbm.at[pl.ds(base, H)], cidx)
            pltpu.async_copy(g_hbm.at[ridx.at[0]], bufa, sema)

            @pl.loop(0, H, step=2)
            def _(t):
                pltpu.async_copy(g_hbm.at[ridx.at[t + 1]], bufb, semb)
                pltpu.make_async_copy(g_hbm.at[ridx.at[t]], bufa, sema).wait()
                pltpu.sync_copy(bufa, acc.at[cidx.at[t]], add=True)
                tn = jnp.minimum(t + 2, H - 1)
                pltpu.async_copy(g_hbm.at[ridx.at[tn]], bufa, sema)
                pltpu.make_async_copy(
                    g_hbm.at[ridx.at[t + 1]], bufb, semb).wait()
                pltpu.sync_copy(bufb, acc.at[cidx.at[t + 1]], add=True)

            pltpu.make_async_copy(g_hbm.at[ridx.at[H - 1]], bufa, sema).wait()

        plsc.subcore_barrier()
        pltpu.sync_copy(acc.at[pl.ds(s * RPT, RPT)],
                        out_hbm.at[c, pl.ds(s * RPT, RPT)])

    return _prop


def _edge_pipeline(gsp, acc, ridx, cidx, bufa, bufb, sga, sgb, ssa, ssb):
    T = IDX_ROWS_PER_TILE
    pltpu.async_copy(gsp.at[ridx.at[0]], bufa, sga)
    pltpu.make_async_copy(gsp.at[ridx.at[0]], bufa, sga).wait()
    pltpu.async_copy(bufa, acc.at[cidx.at[0]], ssa, add=True)
    pltpu.async_copy(gsp.at[ridx.at[1]], bufb, sgb)

    @pl.loop(1, T - 1, step=2)
    def _(t):
        pltpu.make_async_copy(gsp.at[ridx.at[t]], bufb, sgb).wait()
        pltpu.async_copy(bufb, acc.at[cidx.at[t]], ssb, add=True)
        pltpu.make_async_copy(bufa, acc.at[cidx.at[t - 1]], ssa).wait()
        pltpu.async_copy(gsp.at[ridx.at[t + 1]], bufa, sga)
        pltpu.make_async_copy(gsp.at[ridx.at[t + 1]], bufa, sga).wait()
        pltpu.async_copy(bufa, acc.at[cidx.at[t + 1]], ssa, add=True)
        pltpu.make_async_copy(bufb, acc.at[cidx.at[t]], ssb).wait()
        pltpu.async_copy(gsp.at[ridx.at[t + 2]], bufb, sgb)

    pltpu.make_async_copy(gsp.at[ridx.at[T - 1]], bufb, sgb).wait()
    pltpu.async_copy(bufb, acc.at[cidx.at[T - 1]], ssb, add=True)
    pltpu.make_async_copy(bufa, acc.at[cidx.at[T - 2]], ssa).wait()
    pltpu.make_async_copy(bufb, acc.at[cidx.at[T - 1]], ssb).wait()


def _prop_scratch(D):
    return [
        pltpu.VMEM((IDX_ROWS_PER_TILE, 128), jnp.int32),
        pltpu.VMEM((IDX_ROWS_PER_TILE, 128), jnp.int32),
        pltpu.VMEM((128, D), jnp.float32),
        pltpu.VMEM((128, D), jnp.float32),
        pltpu.SemaphoreType.DMA,
        pltpu.SemaphoreType.DMA,
        pltpu.SemaphoreType.DMA,
        pltpu.SemaphoreType.DMA,
        pltpu.VMEM_SHARED((NP, D), jnp.float32),
        pltpu.VMEM_SHARED((NP, D), jnp.float32),
    ]


def _make_prop_local2(D):
    T = IDX_ROWS_PER_TILE

    @functools.partial(
        pl.kernel,
        out_type=jax.ShapeDtypeStruct((NC, 2, NP, D), jnp.float32),
        mesh=_mesh,
        compiler_params=pltpu.CompilerParams(use_tc_tiling_on_sc=False),
        scratch_types=_prop_scratch(D),
    )
    def _prop(g_hbm, row_hbm, col_hbm, out_hbm, ridx, cidx, bufa, bufb,
              sga, sgb, ssa, ssb, gsp, acc):
        c = lax.axis_index("c")
        s = lax.axis_index("s")
        wid = c * NS + s

        pltpu.sync_copy(row_hbm.at[pl.ds(wid * T, T)], ridx)
        pltpu.sync_copy(col_hbm.at[pl.ds(wid * T, T)], cidx)

        for p in range(2):
            pltpu.sync_copy(g_hbm.at[p, pl.ds(s * RPT, RPT)],
                            gsp.at[pl.ds(s * RPT, RPT)])
            pltpu.sync_copy(g_hbm.at[p, pl.ds(s * RPT, RPT)],
                            acc.at[pl.ds(s * RPT, RPT)])
            plsc.subcore_barrier()

            _edge_pipeline(gsp, acc, ridx, cidx, bufa, bufb,
                           sga, sgb, ssa, ssb)

            plsc.subcore_barrier()
            pltpu.sync_copy(acc.at[pl.ds(s * RPT, RPT)],
                            out_hbm.at[c, p, pl.ds(s * RPT, RPT)])
            if p == 0:
                plsc.subcore_barrier()

    return _prop


def _make_prop_local(D):
    T = IDX_ROWS_PER_TILE

    @functools.partial(
        pl.kernel,
        out_type=jax.ShapeDtypeStruct((NC, NP, D), jnp.float32),
        mesh=_mesh,
        compiler_params=pltpu.CompilerParams(
            use_tc_tiling_on_sc=(D % 128 == 0)),
        scratch_types=_prop_scratch(D),
    )
    def _prop(g_hbm, row_hbm, col_hbm, out_hbm, ridx, cidx, bufa, bufb,
              sga, sgb, ssa, ssb, gsp, acc):
        c = lax.axis_index("c")
        s = lax.axis_index("s")
        wid = c * NS + s

        pltpu.sync_copy(row_hbm.at[pl.ds(wid * T, T)], ridx)
        pltpu.sync_copy(col_hbm.at[pl.ds(wid * T, T)], cidx)
        pltpu.sync_copy(g_hbm.at[pl.ds(s * RPT, RPT)],
                        gsp.at[pl.ds(s * RPT, RPT)])
        pltpu.sync_copy(g_hbm.at[pl.ds(s * RPT, RPT)],
                        acc.at[pl.ds(s * RPT, RPT)])
        plsc.subcore_barrier()

        _edge_pipeline(gsp, acc, ridx, cidx, bufa, bufb, sga, sgb, ssa, ssb)

        plsc.subcore_barrier()
        pltpu.sync_copy(acc.at[pl.ds(s * RPT, RPT)],
                        out_hbm.at[c, pl.ds(s * RPT, RPT)])

    return _prop


_prop128 = _make_prop_local2(NHID // 2)
_prop48 = _make_prop_local(D2P)

BLK = 1024


HH = NHID // 2


def _tc_mm_body(x_ref, w_ref, h_ref):
    h_ref[...] = jnp.dot(x_ref[...], w_ref[...],
                         preferred_element_type=jnp.float32)


_tc_mm = pl.pallas_call(
    _tc_mm_body,
    grid=(NP // BLK,),
    in_specs=[
        pl.BlockSpec((BLK, NFEAT), lambda i: (i, 0)),
        pl.BlockSpec((NFEAT, NHID), lambda i: (0, 0)),
    ],
    out_specs=pl.BlockSpec((BLK, NHID), lambda i: (i, 0)),
    out_shape=jax.ShapeDtypeStruct((NP, NHID), jnp.float32),
)


def _tc_scale_body(d0_ref, d1_ref, h_ref, g_ref, dinv_ref):
    deg = d0_ref[...][:, 0:1] + d1_ref[...][:, 0:1] + 1.0
    dinv = lax.rsqrt(deg)
    g = h_ref[...] * dinv
    g_ref[0] = g[:, :HH]
    g_ref[1] = g[:, HH:]
    dinv_ref[...] = dinv


_tc_scale = pl.pallas_call(
    _tc_scale_body,
    grid=(NP // BLK,),
    in_specs=[
        pl.BlockSpec((BLK, L), lambda i: (i, 0)),
        pl.BlockSpec((BLK, L), lambda i: (i, 0)),
        pl.BlockSpec((BLK, NHID), lambda i: (i, 0)),
    ],
    out_specs=[
        pl.BlockSpec((2, BLK, HH), lambda i: (0, i, 0)),
        pl.BlockSpec((BLK, 1), lambda i: (i, 0)),
    ],
    out_shape=[
        jax.ShapeDtypeStruct((2, NP, HH), jnp.float32),
        jax.ShapeDtypeStruct((NP, 1), jnp.float32),
    ],
)


def _tc_mid_body(a_ref, g1_ref, dinv_ref, b1_ref, w2_ref, g2_ref):
    dinv = dinv_ref[...]
    acc = jnp.concatenate(
        [a_ref[0, 0] + a_ref[1, 0], a_ref[0, 1] + a_ref[1, 1]], axis=1)
    g1 = jnp.concatenate([g1_ref[0], g1_ref[1]], axis=1)
    x1 = jnp.maximum(dinv * (acc - g1) + b1_ref[...], 0.0)
    g2_ref[...] = dinv * jnp.dot(
        x1, w2_ref[...], preferred_element_type=jnp.float32)


_tc_mid = pl.pallas_call(
    _tc_mid_body,
    grid=(NP // BLK,),
    in_specs=[
        pl.BlockSpec((NC, 2, BLK, HH), lambda i: (0, 0, i, 0)),
        pl.BlockSpec((2, BLK, HH), lambda i: (0, i, 0)),
        pl.BlockSpec((BLK, 1), lambda i: (i, 0)),
        pl.BlockSpec((1, NHID), lambda i: (0, 0)),
        pl.BlockSpec((NHID, D2P), lambda i: (0, 0)),
    ],
    out_specs=pl.BlockSpec((BLK, D2P), lambda i: (i, 0)),
    out_shape=jax.ShapeDtypeStruct((NP, D2P), jnp.float32),
)


def _tc_post_body(a_ref, g2_ref, dinv_ref, b2_ref, o_ref):
    dinv = dinv_ref[...]
    o_ref[...] = dinv * (a_ref[0] + a_ref[1] - g2_ref[...]) + b2_ref[...]


_tc_post = pl.pallas_call(
    _tc_post_body,
    grid=(NP // BLK,),
    in_specs=[
        pl.BlockSpec((NC, BLK, D2P), lambda i: (0, i, 0)),
        pl.BlockSpec((BLK, D2P), lambda i: (i, 0)),
        pl.BlockSpec((BLK, 1), lambda i: (i, 0)),
        pl.BlockSpec((1, D2P), lambda i: (0, 0)),
    ],
    out_specs=pl.BlockSpec((BLK, D2P), lambda i: (i, 0)),
    out_shape=jax.ShapeDtypeStruct((NP, D2P), jnp.float32),
)


def kernel(x, adj, W1, b1, W2, b2):
    row = adj[0].astype(jnp.int32)
    col = adj[1].astype(jnp.int32)
    pad = E_PAD - E
    rowp = jnp.concatenate(
        [row, jnp.zeros((pad,), jnp.int32)]).reshape(E_PAD // 128, 128)
    colp = jnp.concatenate(
        [col, jnp.full((pad,), N, jnp.int32)]).reshape(E_PAD // 128, 128)
    xp = jnp.pad(x, ((0, NP - N), (0, 0)))

    degp = _sc_degree(colp)
    h1 = _tc_mm(xp, W1)
    g1, dinv = _tc_scale(degp[0], degp[1], h1)
    acc1 = _prop128(g1, rowp, colp)

    W2p = jnp.pad(W2, ((0, 0), (0, D2P - NCLASS)))
    b1r = b1.reshape(1, NHID)
    b2p = jnp.pad(b2, (0, D2P - NCLASS)).reshape(1, D2P)

    g2 = _tc_mid(acc1, g1, dinv, b1r, W2p)
    acc2 = _prop48(g2, rowp, colp)
    out = _tc_post(acc2, g2, dinv, b2p)
    return out[:N, :NCLASS]


# speedup vs baseline: 2.9617x; 1.0952x over previous
import functools

import jax
import jax.numpy as jnp
from jax import lax
from jax.experimental import pallas as pl
from jax.experimental.pallas import tpu as pltpu
from jax.experimental.pallas import tpu_sc as plsc

N = 10000
NFEAT = 128
NHID = 128
NCLASS = 40
E = 320000

NC = 2
NS = 16
L = 16
NW = NC * NS

NCHUNK = E // 128
T0 = 78
NP = 10240
RPT = NP // NS
D2P = 48

_mesh = plsc.VectorSubcoreMesh(core_axis_name="c", subcore_axis_name="s")


def _stage_idx(adj_hbm, idx, sem, base, extra):
    @pl.loop(0, T0)
    def _(t):
        pltpu.async_copy(adj_hbm.at[:, pl.ds((base + t) * 128, 128)],
                         idx.at[pl.ds(2 * t, 2)], sem)

    @pl.when(extra)
    def _():
        pltpu.async_copy(adj_hbm.at[:, pl.ds((base + T0) * 128, 128)],
                         idx.at[pl.ds(2 * T0, 2)], sem)

    @pl.loop(0, T0)
    def _(t):
        pltpu.make_async_copy(adj_hbm.at[:, pl.ds((base + t) * 128, 128)],
                              idx.at[pl.ds(2 * t, 2)], sem).wait()

    @pl.when(extra)
    def _():
        pltpu.make_async_copy(adj_hbm.at[:, pl.ds((base + T0) * 128, 128)],
                              idx.at[pl.ds(2 * T0, 2)], sem).wait()


@functools.partial(
    pl.kernel,
    out_type=jax.ShapeDtypeStruct((NC, NP, L), jnp.float32),
    mesh=_mesh,
    scratch_types=[
        pltpu.VMEM((2 * (T0 + 1), 128), jnp.int32),
        pltpu.VMEM((128, L), jnp.float32),
        pltpu.SemaphoreType.DMA,
        pltpu.SemaphoreType.DMA,
        pltpu.VMEM_SHARED((NP, L), jnp.float32),
    ],
)
def _sc_degree(adj_hbm, out_hbm, idx, buf, sem, ssem, acc):
    c = lax.axis_index("c")
    s = lax.axis_index("s")
    wid = c * NS + s
    base = T0 * wid + jnp.minimum(wid, 4)
    extra = wid < 4

    @pl.loop(0, 128)
    def _(i):
        buf[i, :] = jnp.zeros((L,), jnp.float32)

    @pl.loop(0, RPT // 128)
    def _(z):
        pltpu.sync_copy(buf, acc.at[pl.ds(s * RPT + z * 128, 128)])

    @pl.loop(0, 128)
    def _(i):
        buf[i, :] = jnp.full((L,), 1.0, jnp.float32)

    _stage_idx(adj_hbm, idx, sem, base, extra)
    plsc.subcore_barrier()

    @pl.loop(0, T0)
    def _(t):
        pltpu.async_copy(buf, acc.at[idx.at[2 * t + 1]], ssem, add=True)

    @pl.when(extra)
    def _():
        pltpu.async_copy(buf, acc.at[idx.at[2 * T0 + 1]], ssem, add=True)

    @pl.loop(0, T0)
    def _(t):
        pltpu.make_async_copy(buf, acc.at[idx.at[2 * t + 1]], ssem).wait()

    @pl.when(extra)
    def _():
        pltpu.make_async_copy(buf, acc.at[idx.at[2 * T0 + 1]], ssem).wait()

    plsc.subcore_barrier()
    pltpu.sync_copy(acc.at[pl.ds(s * RPT, RPT)],
                    out_hbm.at[c, pl.ds(s * RPT, RPT)])


def _make_prop(D):
    T = IDX_ROWS_PER_TILE
    H = T // 2

    @functools.partial(
        pl.kernel,
        out_type=jax.ShapeDtypeStruct((NC, NP, D), jnp.float32),
        mesh=_mesh,
        compiler_params=pltpu.CompilerParams(
            use_tc_tiling_on_sc=(D % 128 == 0)),
        scratch_types=[
            pltpu.VMEM((H, 128), jnp.int32),
            pltpu.VMEM((H, 128), jnp.int32),
            pltpu.VMEM((128, D), jnp.float32),
            pltpu.VMEM((128, D), jnp.float32),
            pltpu.SemaphoreType.DMA,
            pltpu.SemaphoreType.DMA,
            pltpu.VMEM_SHARED((NP, D), jnp.float32),
        ],
    )
    def _prop(g_hbm, row_hbm, col_hbm, out_hbm, ridx, cidx, bufa, bufb,
              sema, semb, acc):
        c = lax.axis_index("c")
        s = lax.axis_index("s")
        wid = c * NS + s

        pltpu.sync_copy(g_hbm.at[pl.ds(s * RPT, RPT)],
                        acc.at[pl.ds(s * RPT, RPT)])
        plsc.subcore_barrier()

        for h in range(2):
            base = wid * T + h * H
            pltpu.sync_copy(row_hbm.at[pl.ds(base, H)], ridx)
            pltpu.sync_copy(col_hbm.at[pl.ds(base, H)], cidx)
            pltpu.async_copy(g_hbm.at[ridx.at[0]], bufa, sema)

            @pl.loop(0, H, step=2)
            def _(t):
                pltpu.async_copy(g_hbm.at[ridx.at[t + 1]], bufb, semb)
                pltpu.make_async_copy(g_hbm.at[ridx.at[t]], bufa, sema).wait()
                pltpu.sync_copy(bufa, acc.at[cidx.at[t]], add=True)
                tn = jnp.minimum(t + 2, H - 1)
                pltpu.async_copy(g_hbm.at[ridx.at[tn]], bufa, sema)
                pltpu.make_async_copy(
                    g_hbm.at[ridx.at[t + 1]], bufb, semb).wait()
                pltpu.sync_copy(bufb, acc.at[cidx.at[t + 1]], add=True)

            pltpu.make_async_copy(g_hbm.at[ridx.at[H - 1]], bufa, sema).wait()

        plsc.subcore_barrier()
        pltpu.sync_copy(acc.at[pl.ds(s * RPT, RPT)],
                        out_hbm.at[c, pl.ds(s * RPT, RPT)])

    return _prop


def _edge_pipeline(gsp, acc, idx, bufa, bufb, sga, sgb, ssa, ssb, extra):
    pltpu.async_copy(gsp.at[idx.at[0]], bufa, sga)
    pltpu.make_async_copy(gsp.at[idx.at[0]], bufa, sga).wait()
    pltpu.async_copy(bufa, acc.at[idx.at[1]], ssa, add=True)
    pltpu.async_copy(gsp.at[idx.at[2]], bufb, sgb)

    @pl.loop(1, T0 - 1, step=2)
    def _(t):
        pltpu.make_async_copy(gsp.at[idx.at[2 * t]], bufb, sgb).wait()
        pltpu.async_copy(bufb, acc.at[idx.at[2 * t + 1]], ssb, add=True)
        pltpu.make_async_copy(bufa, acc.at[idx.at[2 * t - 1]], ssa).wait()
        pltpu.async_copy(gsp.at[idx.at[2 * t + 2]], bufa, sga)
        pltpu.make_async_copy(gsp.at[idx.at[2 * t + 2]], bufa, sga).wait()
        pltpu.async_copy(bufa, acc.at[idx.at[2 * t + 3]], ssa, add=True)
        pltpu.make_async_copy(bufb, acc.at[idx.at[2 * t + 1]], ssb).wait()
        pltpu.async_copy(gsp.at[idx.at[2 * t + 4]], bufb, sgb)

    pltpu.make_async_copy(gsp.at[idx.at[2 * T0 - 2]], bufb, sgb).wait()
    pltpu.async_copy(bufb, acc.at[idx.at[2 * T0 - 1]], ssb, add=True)
    pltpu.make_async_copy(bufa, acc.at[idx.at[2 * T0 - 3]], ssa).wait()
    pltpu.make_async_copy(bufb, acc.at[idx.at[2 * T0 - 1]], ssb).wait()

    @pl.when(extra)
    def _():
        pltpu.async_copy(gsp.at[idx.at[2 * T0]], bufa, sga)
        pltpu.make_async_copy(gsp.at[idx.at[2 * T0]], bufa, sga).wait()
        pltpu.sync_copy(bufa, acc.at[idx.at[2 * T0 + 1]], add=True)


def _prop_scratch(D):
    return [
        pltpu.VMEM((2 * (T0 + 1), 128), jnp.int32),
        pltpu.VMEM((128, D), jnp.float32),
        pltpu.VMEM((128, D), jnp.float32),
        pltpu.SemaphoreType.DMA,
        pltpu.SemaphoreType.DMA,
        pltpu.SemaphoreType.DMA,
        pltpu.SemaphoreType.DMA,
        pltpu.SemaphoreType.DMA,
        pltpu.VMEM_SHARED((NP, D), jnp.float32),
        pltpu.VMEM_SHARED((NP, D), jnp.float32),
    ]


def _make_prop_local2(D):
    @functools.partial(
        pl.kernel,
        out_type=jax.ShapeDtypeStruct((NC, 2, NP, D), jnp.float32),
        mesh=_mesh,
        compiler_params=pltpu.CompilerParams(use_tc_tiling_on_sc=False),
        scratch_types=_prop_scratch(D),
    )
    def _prop(g_hbm, adj_hbm, out_hbm, idx, bufa, bufb,
              sia, sga, sgb, ssa, ssb, gsp, acc):
        c = lax.axis_index("c")
        s = lax.axis_index("s")
        wid = c * NS + s
        base = T0 * wid + jnp.minimum(wid, 4)
        extra = wid < 4

        _stage_idx(adj_hbm, idx, sia, base, extra)

        for p in range(2):
            pltpu.sync_copy(g_hbm.at[p, pl.ds(s * RPT, RPT)],
                            gsp.at[pl.ds(s * RPT, RPT)])
            pltpu.sync_copy(g_hbm.at[p, pl.ds(s * RPT, RPT)],
                            acc.at[pl.ds(s * RPT, RPT)])
            plsc.subcore_barrier()

            _edge_pipeline(gsp, acc, idx, bufa, bufb,
                           sga, sgb, ssa, ssb, extra)

            plsc.subcore_barrier()
            pltpu.sync_copy(acc.at[pl.ds(s * RPT, RPT)],
                            out_hbm.at[c, p, pl.ds(s * RPT, RPT)])
            if p == 0:
                plsc.subcore_barrier()

    return _prop


def _make_prop_local(D):
    @functools.partial(
        pl.kernel,
        out_type=jax.ShapeDtypeStruct((NC, NP, D), jnp.float32),
        mesh=_mesh,
        compiler_params=pltpu.CompilerParams(
            use_tc_tiling_on_sc=(D % 128 == 0)),
        scratch_types=_prop_scratch(D),
    )
    def _prop(g_hbm, adj_hbm, out_hbm, idx, bufa, bufb,
              sia, sga, sgb, ssa, ssb, gsp, acc):
        c = lax.axis_index("c")
        s = lax.axis_index("s")
        wid = c * NS + s
        base = T0 * wid + jnp.minimum(wid, 4)
        extra = wid < 4

        _stage_idx(adj_hbm, idx, sia, base, extra)
        pltpu.sync_copy(g_hbm.at[pl.ds(s * RPT, RPT)],
                        gsp.at[pl.ds(s * RPT, RPT)])
        pltpu.sync_copy(g_hbm.at[pl.ds(s * RPT, RPT)],
                        acc.at[pl.ds(s * RPT, RPT)])
        plsc.subcore_barrier()

        _edge_pipeline(gsp, acc, idx, bufa, bufb, sga, sgb, ssa, ssb, extra)

        plsc.subcore_barrier()
        pltpu.sync_copy(acc.at[pl.ds(s * RPT, RPT)],
                        out_hbm.at[c, pl.ds(s * RPT, RPT)])

    return _prop


_prop128 = _make_prop_local2(NHID // 2)
_prop48 = _make_prop_local(D2P)

BLK = 1024


HH = NHID // 2


def _tc_mm_body(x_ref, w_ref, h_ref):
    h_ref[...] = jnp.dot(x_ref[...], w_ref[...],
                         preferred_element_type=jnp.float32)


_tc_mm = pl.pallas_call(
    _tc_mm_body,
    grid=(NP // BLK,),
    in_specs=[
        pl.BlockSpec((BLK, NFEAT), lambda i: (i, 0)),
        pl.BlockSpec((NFEAT, NHID), lambda i: (0, 0)),
    ],
    out_specs=pl.BlockSpec((BLK, NHID), lambda i: (i, 0)),
    out_shape=jax.ShapeDtypeStruct((NP, NHID), jnp.float32),
)


def _tc_scale_body(d0_ref, d1_ref, h_ref, g_ref, dinv_ref):
    deg = d0_ref[...][:, 0:1] + d1_ref[...][:, 0:1] + 1.0
    dinv = lax.rsqrt(deg)
    g = h_ref[...] * dinv
    g_ref[0] = g[:, :HH]
    g_ref[1] = g[:, HH:]
    dinv_ref[...] = dinv


_tc_scale = pl.pallas_call(
    _tc_scale_body,
    grid=(NP // BLK,),
    in_specs=[
        pl.BlockSpec((BLK, L), lambda i: (i, 0)),
        pl.BlockSpec((BLK, L), lambda i: (i, 0)),
        pl.BlockSpec((BLK, NHID), lambda i: (i, 0)),
    ],
    out_specs=[
        pl.BlockSpec((2, BLK, HH), lambda i: (0, i, 0)),
        pl.BlockSpec((BLK, 1), lambda i: (i, 0)),
    ],
    out_shape=[
        jax.ShapeDtypeStruct((2, NP, HH), jnp.float32),
        jax.ShapeDtypeStruct((NP, 1), jnp.float32),
    ],
)


def _tc_mid_body(a_ref, g1_ref, dinv_ref, b1_ref, w2_ref, g2_ref):
    dinv = dinv_ref[...]
    acc = jnp.concatenate(
        [a_ref[0, 0] + a_ref[1, 0], a_ref[0, 1] + a_ref[1, 1]], axis=1)
    g1 = jnp.concatenate([g1_ref[0], g1_ref[1]], axis=1)
    x1 = jnp.maximum(dinv * (acc - g1) + b1_ref[...], 0.0)
    g2_ref[...] = dinv * jnp.dot(
        x1, w2_ref[...], preferred_element_type=jnp.float32)


_tc_mid = pl.pallas_call(
    _tc_mid_body,
    grid=(NP // BLK,),
    in_specs=[
        pl.BlockSpec((NC, 2, BLK, HH), lambda i: (0, 0, i, 0)),
        pl.BlockSpec((2, BLK, HH), lambda i: (0, i, 0)),
        pl.BlockSpec((BLK, 1), lambda i: (i, 0)),
        pl.BlockSpec((1, NHID), lambda i: (0, 0)),
        pl.BlockSpec((NHID, D2P), lambda i: (0, 0)),
    ],
    out_specs=pl.BlockSpec((BLK, D2P), lambda i: (i, 0)),
    out_shape=jax.ShapeDtypeStruct((NP, D2P), jnp.float32),
)


def _tc_post_body(a_ref, g2_ref, dinv_ref, b2_ref, o_ref):
    dinv = dinv_ref[...]
    o_ref[...] = dinv * (a_ref[0] + a_ref[1] - g2_ref[...]) + b2_ref[...]


_tc_post = pl.pallas_call(
    _tc_post_body,
    grid=(NP // BLK,),
    in_specs=[
        pl.BlockSpec((NC, BLK, D2P), lambda i: (0, i, 0)),
        pl.BlockSpec((BLK, D2P), lambda i: (i, 0)),
        pl.BlockSpec((BLK, 1), lambda i: (i, 0)),
        pl.BlockSpec((1, D2P), lambda i: (0, 0)),
    ],
    out_specs=pl.BlockSpec((BLK, D2P), lambda i: (i, 0)),
    out_shape=jax.ShapeDtypeStruct((NP, D2P), jnp.float32),
)


def kernel(x, adj, W1, b1, W2, b2):
    adjc = adj.astype(jnp.int32)
    xp = jnp.pad(x, ((0, NP - N), (0, 0)))

    degp = _sc_degree(adjc)
    h1 = _tc_mm(xp, W1)
    g1, dinv = _tc_scale(degp[0], degp[1], h1)
    acc1 = _prop128(g1, adjc)

    W2p = jnp.pad(W2, ((0, 0), (0, D2P - NCLASS)))
    b1r = b1.reshape(1, NHID)
    b2p = jnp.pad(b2, (0, D2P - NCLASS)).reshape(1, D2P)

    g2 = _tc_mid(acc1, g1, dinv, b1r, W2p)
    acc2 = _prop48(g2, adjc)
    out = _tc_post(acc2, g2, dinv, b2p)
    return out[:N, :NCLASS]


# speedup vs baseline: 2.9630x; 1.0005x over previous
import functools

import jax
import jax.numpy as jnp
from jax import lax
from jax.experimental import pallas as pl
from jax.experimental.pallas import tpu as pltpu
from jax.experimental.pallas import tpu_sc as plsc

N = 10000
NFEAT = 128
NHID = 128
NCLASS = 40
E = 320000

NC = 2
NS = 16
L = 16
NW = NC * NS

NCHUNK = E // 128
T0 = 78
NP = 10240
RPT = NP // NS
D2P = 48

_mesh = plsc.VectorSubcoreMesh(core_axis_name="c", subcore_axis_name="s")


def _stage_idx(adj_hbm, idx, sem, base, extra):
    @pl.loop(0, T0)
    def _(t):
        pltpu.async_copy(adj_hbm.at[:, pl.ds((base + t) * 128, 128)],
                         idx.at[pl.ds(2 * t, 2)], sem)

    @pl.when(extra)
    def _():
        pltpu.async_copy(adj_hbm.at[:, pl.ds((base + T0) * 128, 128)],
                         idx.at[pl.ds(2 * T0, 2)], sem)

    @pl.loop(0, T0)
    def _(t):
        pltpu.make_async_copy(adj_hbm.at[:, pl.ds((base + t) * 128, 128)],
                              idx.at[pl.ds(2 * t, 2)], sem).wait()

    @pl.when(extra)
    def _():
        pltpu.make_async_copy(adj_hbm.at[:, pl.ds((base + T0) * 128, 128)],
                              idx.at[pl.ds(2 * T0, 2)], sem).wait()


@functools.partial(
    pl.kernel,
    out_type=jax.ShapeDtypeStruct((NC, NP, L), jnp.float32),
    mesh=_mesh,
    scratch_types=[
        pltpu.VMEM((2 * (T0 + 1), 128), jnp.int32),
        pltpu.VMEM((128, L), jnp.float32),
        pltpu.SemaphoreType.DMA,
        pltpu.SemaphoreType.DMA,
        pltpu.VMEM_SHARED((NP, L), jnp.float32),
    ],
)
def _sc_degree(adj_hbm, out_hbm, idx, buf, sem, ssem, acc):
    c = lax.axis_index("c")
    s = lax.axis_index("s")
    wid = c * NS + s
    base = T0 * wid + jnp.minimum(wid, 4)
    extra = wid < 4

    @pl.loop(0, 128)
    def _(i):
        buf[i, :] = jnp.zeros((L,), jnp.float32)

    @pl.loop(0, RPT // 128)
    def _(z):
        pltpu.sync_copy(buf, acc.at[pl.ds(s * RPT + z * 128, 128)])

    @pl.loop(0, 128)
    def _(i):
        buf[i, :] = jnp.full((L,), 1.0, jnp.float32)

    _stage_idx(adj_hbm, idx, sem, base, extra)
    plsc.subcore_barrier()

    @pl.loop(0, T0)
    def _(t):
        pltpu.async_copy(buf, acc.at[idx.at[2 * t + 1]], ssem, add=True)

    @pl.when(extra)
    def _():
        pltpu.async_copy(buf, acc.at[idx.at[2 * T0 + 1]], ssem, add=True)

    @pl.loop(0, T0)
    def _(t):
        pltpu.make_async_copy(buf, acc.at[idx.at[2 * t + 1]], ssem).wait()

    @pl.when(extra)
    def _():
        pltpu.make_async_copy(buf, acc.at[idx.at[2 * T0 + 1]], ssem).wait()

    plsc.subcore_barrier()
    pltpu.sync_copy(acc.at[pl.ds(s * RPT, RPT)],
                    out_hbm.at[c, pl.ds(s * RPT, RPT)])


def _edge_pipeline(gsp, acc, idx, bufa, bufb, sga, sgb, ssa, ssb, extra):
    pltpu.async_copy(gsp.at[idx.at[0]], bufa, sga)
    pltpu.make_async_copy(gsp.at[idx.at[0]], bufa, sga).wait()
    pltpu.async_copy(bufa, acc.at[idx.at[1]], ssa, add=True)
    pltpu.async_copy(gsp.at[idx.at[2]], bufb, sgb)

    @pl.loop(1, T0 - 1, step=2)
    def _(t):
        pltpu.make_async_copy(gsp.at[idx.at[2 * t]], bufb, sgb).wait()
        pltpu.async_copy(bufb, acc.at[idx.at[2 * t + 1]], ssb, add=True)
        pltpu.make_async_copy(bufa, acc.at[idx.at[2 * t - 1]], ssa).wait()
        pltpu.async_copy(gsp.at[idx.at[2 * t + 2]], bufa, sga)
        pltpu.make_async_copy(gsp.at[idx.at[2 * t + 2]], bufa, sga).wait()
        pltpu.async_copy(bufa, acc.at[idx.at[2 * t + 3]], ssa, add=True)
        pltpu.make_async_copy(bufb, acc.at[idx.at[2 * t + 1]], ssb).wait()
        pltpu.async_copy(gsp.at[idx.at[2 * t + 4]], bufb, sgb)

    pltpu.make_async_copy(gsp.at[idx.at[2 * T0 - 2]], bufb, sgb).wait()
    pltpu.async_copy(bufb, acc.at[idx.at[2 * T0 - 1]], ssb, add=True)
    pltpu.make_async_copy(bufa, acc.at[idx.at[2 * T0 - 3]], ssa).wait()
    pltpu.make_async_copy(bufb, acc.at[idx.at[2 * T0 - 1]], ssb).wait()

    @pl.when(extra)
    def _():
        pltpu.async_copy(gsp.at[idx.at[2 * T0]], bufa, sga)
        pltpu.make_async_copy(gsp.at[idx.at[2 * T0]], bufa, sga).wait()
        pltpu.sync_copy(bufa, acc.at[idx.at[2 * T0 + 1]], add=True)


def _prop_scratch(D):
    return [
        pltpu.VMEM((2 * (T0 + 1), 128), jnp.int32),
        pltpu.VMEM((128, D), jnp.float32),
        pltpu.VMEM((128, D), jnp.float32),
        pltpu.SemaphoreType.DMA,
        pltpu.SemaphoreType.DMA,
        pltpu.SemaphoreType.DMA,
        pltpu.SemaphoreType.DMA,
        pltpu.SemaphoreType.DMA,
        pltpu.VMEM_SHARED((NP, D), jnp.float32),
        pltpu.VMEM_SHARED((NP, D), jnp.float32),
    ]


def _make_prop_local2(D):
    @functools.partial(
        pl.kernel,
        out_type=jax.ShapeDtypeStruct((NC, 2, NP, D), jnp.float32),
        mesh=_mesh,
        compiler_params=pltpu.CompilerParams(use_tc_tiling_on_sc=False),
        scratch_types=_prop_scratch(D),
    )
    def _prop(g_hbm, adj_hbm, out_hbm, idx, bufa, bufb,
              sia, sga, sgb, ssa, ssb, gsp, acc):
        c = lax.axis_index("c")
        s = lax.axis_index("s")
        wid = c * NS + s
        base = T0 * wid + jnp.minimum(wid, 4)
        extra = wid < 4

        _stage_idx(adj_hbm, idx, sia, base, extra)

        for p in range(2):
            pltpu.sync_copy(g_hbm.at[p, pl.ds(s * RPT, RPT)],
                            gsp.at[pl.ds(s * RPT, RPT)])
            pltpu.sync_copy(g_hbm.at[p, pl.ds(s * RPT, RPT)],
                            acc.at[pl.ds(s * RPT, RPT)])
            plsc.subcore_barrier()

            _edge_pipeline(gsp, acc, idx, bufa, bufb,
                           sga, sgb, ssa, ssb, extra)

            plsc.subcore_barrier()
            pltpu.sync_copy(acc.at[pl.ds(s * RPT, RPT)],
                            out_hbm.at[c, p, pl.ds(s * RPT, RPT)])
            if p == 0:
                plsc.subcore_barrier()

    return _prop


def _make_prop_local(D):
    @functools.partial(
        pl.kernel,
        out_type=jax.ShapeDtypeStruct((NC, NP, D), jnp.float32),
        mesh=_mesh,
        compiler_params=pltpu.CompilerParams(
            use_tc_tiling_on_sc=(D % 128 == 0)),
        scratch_types=_prop_scratch(D),
    )
    def _prop(g_hbm, adj_hbm, out_hbm, idx, bufa, bufb,
              sia, sga, sgb, ssa, ssb, gsp, acc):
        c = lax.axis_index("c")
        s = lax.axis_index("s")
        wid = c * NS + s
        base = T0 * wid + jnp.minimum(wid, 4)
        extra = wid < 4

        _stage_idx(adj_hbm, idx, sia, base, extra)
        pltpu.sync_copy(g_hbm.at[pl.ds(s * RPT, RPT)],
                        gsp.at[pl.ds(s * RPT, RPT)])
        pltpu.sync_copy(g_hbm.at[pl.ds(s * RPT, RPT)],
                        acc.at[pl.ds(s * RPT, RPT)])
        plsc.subcore_barrier()

        _edge_pipeline(gsp, acc, idx, bufa, bufb, sga, sgb, ssa, ssb, extra)

        plsc.subcore_barrier()
        pltpu.sync_copy(acc.at[pl.ds(s * RPT, RPT)],
                        out_hbm.at[c, pl.ds(s * RPT, RPT)])

    return _prop


_prop128 = _make_prop_local2(NHID // 2)
_prop48 = _make_prop_local(D2P)

BLK = 1024


HH = NHID // 2


def _tc_mm_body(x_ref, w_ref, h_ref):
    h_ref[...] = jnp.dot(x_ref[...], w_ref[...],
                         preferred_element_type=jnp.float32)


_tc_mm = pl.pallas_call(
    _tc_mm_body,
    grid=(NP // BLK,),
    in_specs=[
        pl.BlockSpec((BLK, NFEAT), lambda i: (i, 0)),
        pl.BlockSpec((NFEAT, NHID), lambda i: (0, 0)),
    ],
    out_specs=pl.BlockSpec((BLK, NHID), lambda i: (i, 0)),
    out_shape=jax.ShapeDtypeStruct((NP, NHID), jnp.float32),
)


def _tc_scale_body(d0_ref, d1_ref, h_ref, g_ref, dinv_ref):
    deg = d0_ref[...][:, 0:1] + d1_ref[...][:, 0:1] + 1.0
    dinv = lax.rsqrt(deg)
    g = h_ref[...] * dinv
    g_ref[0] = g[:, :HH]
    g_ref[1] = g[:, HH:]
    dinv_ref[...] = dinv


_tc_scale = pl.pallas_call(
    _tc_scale_body,
    grid=(NP // BLK,),
    in_specs=[
        pl.BlockSpec((BLK, L), lambda i: (i, 0)),
        pl.BlockSpec((BLK, L), lambda i: (i, 0)),
        pl.BlockSpec((BLK, NHID), lambda i: (i, 0)),
    ],
    out_specs=[
        pl.BlockSpec((2, BLK, HH), lambda i: (0, i, 0)),
        pl.BlockSpec((BLK, 1), lambda i: (i, 0)),
    ],
    out_shape=[
        jax.ShapeDtypeStruct((2, NP, HH), jnp.float32),
        jax.ShapeDtypeStruct((NP, 1), jnp.float32),
    ],
)


def _tc_mid_body(a_ref, g1_ref, dinv_ref, b1_ref, w2_ref, g2_ref):
    dinv = dinv_ref[...]
    acc = jnp.concatenate(
        [a_ref[0, 0] + a_ref[1, 0], a_ref[0, 1] + a_ref[1, 1]], axis=1)
    g1 = jnp.concatenate([g1_ref[0], g1_ref[1]], axis=1)
    x1 = jnp.maximum(dinv * (acc - g1) + b1_ref[...], 0.0)
    g2_ref[...] = dinv * jnp.dot(
        x1, w2_ref[...], preferred_element_type=jnp.float32)


_tc_mid = pl.pallas_call(
    _tc_mid_body,
    grid=(NP // BLK,),
    in_specs=[
        pl.BlockSpec((NC, 2, BLK, HH), lambda i: (0, 0, i, 0)),
        pl.BlockSpec((2, BLK, HH), lambda i: (0, i, 0)),
        pl.BlockSpec((BLK, 1), lambda i: (i, 0)),
        pl.BlockSpec((1, NHID), lambda i: (0, 0)),
        pl.BlockSpec((NHID, D2P), lambda i: (0, 0)),
    ],
    out_specs=pl.BlockSpec((BLK, D2P), lambda i: (i, 0)),
    out_shape=jax.ShapeDtypeStruct((NP, D2P), jnp.float32),
)


def _tc_post_body(a_ref, g2_ref, dinv_ref, b2_ref, o_ref):
    dinv = dinv_ref[...]
    o_ref[...] = dinv * (a_ref[0] + a_ref[1] - g2_ref[...]) + b2_ref[...]


_tc_post = pl.pallas_call(
    _tc_post_body,
    grid=(NP // BLK,),
    in_specs=[
        pl.BlockSpec((NC, BLK, D2P), lambda i: (0, i, 0)),
        pl.BlockSpec((BLK, D2P), lambda i: (i, 0)),
        pl.BlockSpec((BLK, 1), lambda i: (i, 0)),
        pl.BlockSpec((1, D2P), lambda i: (0, 0)),
    ],
    out_specs=pl.BlockSpec((BLK, D2P), lambda i: (i, 0)),
    out_shape=jax.ShapeDtypeStruct((NP, D2P), jnp.float32),
)


def kernel(x, adj, W1, b1, W2, b2):
    adjc = adj.astype(jnp.int32)
    xp = jnp.pad(x, ((0, NP - N), (0, 0)))

    degp = _sc_degree(adjc)
    h1 = _tc_mm(xp, W1)
    g1, dinv = _tc_scale(degp[0], degp[1], h1)
    acc1 = _prop128(g1, adjc)

    W2p = jnp.pad(W2, ((0, 0), (0, D2P - NCLASS)))
    b1r = b1.reshape(1, NHID)
    b2p = jnp.pad(b2, (0, D2P - NCLASS)).reshape(1, D2P)

    g2 = _tc_mid(acc1, g1, dinv, b1r, W2p)
    acc2 = _prop48(g2, adjc)
    out = _tc_post(acc2, g2, dinv, b2p)
    return out[:N, :NCLASS]
